# Initial kernel scaffold; baseline (speedup 1.0000x reference)
import jax
import jax.numpy as jnp
from jax.experimental import pallas as pl


def kernel(x, edge_index, edge_weight, batch, W1, b1, W3, b3, fc1W, fc1b, fc2W, fc2b, fc3W, fc3b, bn1g, bn1b, bn2g, bn2b):
    raise NotImplementedError("write your pallas kernel here")


# speedup vs baseline: 41.5945x; 41.5945x over previous
import jax
import jax.numpy as jnp
from jax import lax
from jax.experimental import pallas as pl
from jax.experimental.pallas import tpu as pltpu
from jax.experimental.pallas import tpu_sc as plsc

B = 112
ROIS = 90
RP = 96
DEG = 32
EPG = ROIS * DEG
H0, H1, H3 = 128, 256, 256
D2, D3, C = 64, 16, 2
NW = 32
AW = RP * RP


def _sc_adj_body(src_hbm, dst_hbm, w_hbm, out_hbm, a_v, src_v, dst_v, w_v):
    wid = lax.axis_index("s") * 2 + lax.axis_index("c")
    for t in range(4):
        g = t * NW + wid

        @pl.when(g < B)
        def _():
            def zero_body(i, _):
                a_v[pl.ds(i * 16, 16)] = jnp.zeros((16,), jnp.float32)
                return 0

            lax.fori_loop(0, AW // 16, zero_body, 0)

            eoff = g * EPG
            pltpu.sync_copy(src_hbm.at[pl.ds(eoff, EPG)], src_v)
            pltpu.sync_copy(dst_hbm.at[pl.ds(eoff, EPG)], dst_v)
            pltpu.sync_copy(w_hbm.at[pl.ds(eoff, EPG)], w_v)

            base = g * ROIS

            def edge_body(i, _):
                s16 = src_v[pl.ds(i * 16, 16)]
                d16 = dst_v[pl.ds(i * 16, 16)]
                w16 = w_v[pl.ds(i * 16, 16)]
                idx = (d16 - base) * RP + (s16 - base)
                plsc.addupdate_scatter(a_v, [idx], w16)
                return 0

            lax.fori_loop(0, EPG // 16, edge_body, 0)

            pltpu.sync_copy(a_v, out_hbm.at[g])


def _build_adjacency(src, dst, w):
    mesh = plsc.VectorSubcoreMesh(core_axis_name="c", subcore_axis_name="s")
    kern = pl.kernel(
        _sc_adj_body,
        out_type=jax.ShapeDtypeStruct((B, AW), jnp.float32),
        mesh=mesh,
        compiler_params=pltpu.CompilerParams(needs_layout_passes=False),
        scratch_types=[
            pltpu.VMEM((AW,), jnp.float32),
            pltpu.VMEM((EPG,), jnp.int32),
            pltpu.VMEM((EPG,), jnp.int32),
            pltpu.VMEM((EPG,), jnp.float32),
        ],
    )
    return kern(src, dst, w)


def _conv_body(a_ref, x_ref, w1_ref, b1_ref, w3_ref, b3_ref, h1_ref, h3_ref):
    A = a_ref[0]
    rowsum = jnp.sum(A, axis=1, keepdims=True)
    node = lax.broadcasted_iota(jnp.int32, (RP, 1), 0)
    deg = rowsum + jnp.where(node < ROIS, 1.0, 0.0)
    dinv = jnp.where(deg > 0, lax.rsqrt(deg), 0.0)
    r_i = lax.broadcasted_iota(jnp.int32, (RP, RP), 0)
    c_i = lax.broadcasted_iota(jnp.int32, (RP, RP), 1)
    An = A + jnp.where(r_i == c_i, 1.0, 0.0)

    xg = x_ref[0]
    xw1 = jnp.dot(xg, w1_ref[...], preferred_element_type=jnp.float32)
    h1 = jax.nn.relu(
        dinv * jnp.dot(An, dinv * xw1, preferred_element_type=jnp.float32)
        + b1_ref[...])
    h1_ref[:, 0, 0, :] = h1[:ROIS]
    xw2 = jnp.dot(h1, w3_ref[...], preferred_element_type=jnp.float32)
    h3 = jax.nn.relu(
        dinv * jnp.dot(An, dinv * xw2, preferred_element_type=jnp.float32)
        + b3_ref[...])
    h3_ref[:, 0, 0, :] = h3[:ROIS]


def _gcn_layers(A, xp, W1, b1, W3, b3):
    h_shape = jax.ShapeDtypeStruct((ROIS, B, 1, H1), jnp.float32)
    return pl.pallas_call(
        _conv_body,
        grid=(B,),
        in_specs=[
            pl.BlockSpec((1, RP, RP), lambda g: (g, 0, 0)),
            pl.BlockSpec((1, RP, H0), lambda g: (g, 0, 0)),
            pl.BlockSpec((H0, H1), lambda g: (0, 0)),
            pl.BlockSpec((1, H1), lambda g: (0, 0)),
            pl.BlockSpec((H1, H3), lambda g: (0, 0)),
            pl.BlockSpec((1, H3), lambda g: (0, 0)),
        ],
        out_specs=[
            pl.BlockSpec((ROIS, 1, 1, H1), lambda g: (0, g, 0, 0)),
            pl.BlockSpec((ROIS, 1, 1, H3), lambda g: (0, g, 0, 0)),
        ],
        out_shape=[h_shape, h_shape],
    )(A, xp, W1, b1.reshape(1, -1), W3, b3.reshape(1, -1))


def _head_body(h1_ref, h3_ref, wt_ref, wb_ref, fc1b_ref, fc2w_ref, fc2b_ref,
               fc3w_ref, fc3b_ref, g1_ref, bb1_ref, g2_ref, bb2_ref,
               out_ref, acc):
    r = pl.program_id(0)

    @pl.when(r == 0)
    def _():
        acc[...] = jnp.zeros_like(acc)

    h1r = h1_ref[0, :, 0, :]
    h3r = h3_ref[0, :, 0, :]
    acc[...] += (jnp.dot(h1r, wt_ref[0], preferred_element_type=jnp.float32)
                 + jnp.dot(h3r, wb_ref[0], preferred_element_type=jnp.float32))

    @pl.when(r == ROIS - 1)
    def _():
        y = jax.nn.relu(acc[...] + fc1b_ref[...])
        mu = jnp.mean(y, axis=0, keepdims=True)
        var = jnp.mean((y - mu) ** 2, axis=0, keepdims=True)
        y = g1_ref[...] * (y - mu) / jnp.sqrt(var + 1e-5) + bb1_ref[...]
        y = jax.nn.relu(
            jnp.dot(y, fc2w_ref[...], preferred_element_type=jnp.float32)
            + fc2b_ref[...])
        mu2 = jnp.mean(y, axis=0, keepdims=True)
        var2 = jnp.mean((y - mu2) ** 2, axis=0, keepdims=True)
        y = g2_ref[...] * (y - mu2) / jnp.sqrt(var2 + 1e-5) + bb2_ref[...]
        z = (jnp.dot(y, fc3w_ref[...], preferred_element_type=jnp.float32)
             + fc3b_ref[...])
        m = jnp.max(z, axis=1, keepdims=True)
        lse = m + jnp.log(jnp.sum(jnp.exp(z - m), axis=1, keepdims=True))
        out_ref[...] = z - lse


def _fc_head(h1, h3, wt, wb, fc1b, fc2W, fc2b, fc3W, fc3b,
             bn1g, bn1b, bn2g, bn2b):
    return pl.pallas_call(
        _head_body,
        grid=(ROIS,),
        in_specs=[
            pl.BlockSpec((1, B, 1, H1), lambda r: (r, 0, 0, 0)),
            pl.BlockSpec((1, B, 1, H3), lambda r: (r, 0, 0, 0)),
            pl.BlockSpec((1, H1, D2), lambda r: (r, 0, 0)),
            pl.BlockSpec((1, H3, D2), lambda r: (r, 0, 0)),
            pl.BlockSpec((1, D2), lambda r: (0, 0)),
            pl.BlockSpec((D2, D3), lambda r: (0, 0)),
            pl.BlockSpec((1, D3), lambda r: (0, 0)),
            pl.BlockSpec((D3, C), lambda r: (0, 0)),
            pl.BlockSpec((1, C), lambda r: (0, 0)),
            pl.BlockSpec((1, D2), lambda r: (0, 0)),
            pl.BlockSpec((1, D2), lambda r: (0, 0)),
            pl.BlockSpec((1, D3), lambda r: (0, 0)),
            pl.BlockSpec((1, D3), lambda r: (0, 0)),
        ],
        out_specs=pl.BlockSpec((B, C), lambda r: (0, 0)),
        out_shape=jax.ShapeDtypeStruct((B, C), jnp.float32),
        scratch_shapes=[pltpu.VMEM((B, D2), jnp.float32)],
    )(h1, h3, wt, wb, fc1b.reshape(1, -1), fc2W, fc2b.reshape(1, -1),
      fc3W, fc3b.reshape(1, -1), bn1g.reshape(1, -1), bn1b.reshape(1, -1),
      bn2g.reshape(1, -1), bn2b.reshape(1, -1))


def kernel(x, edge_index, edge_weight, batch, W1, b1, W3, b3,
           fc1W, fc1b, fc2W, fc2b, fc3W, fc3b, bn1g, bn1b, bn2g, bn2b):
    src = edge_index[0]
    dst = edge_index[1]
    adj = _build_adjacency(src, dst, edge_weight)
    A = adj.reshape(B, RP, RP)
    xp = jnp.pad(x.reshape(B, ROIS, H0), ((0, 0), (0, RP - ROIS), (0, 0)))
    h1, h3 = _gcn_layers(A, xp, W1, b1, W3, b3)
    wt = fc1W[:ROIS * H1].reshape(ROIS, H1, D2)
    wb = fc1W[ROIS * H1:].reshape(ROIS, H3, D2)
    return _fc_head(h1, h3, wt, wb, fc1b, fc2W, fc2b, fc3W, fc3b,
                    bn1g, bn1b, bn2g, bn2b)


# speedup vs baseline: 60.0415x; 1.4435x over previous
import jax
import jax.numpy as jnp
from jax import lax
from jax.experimental import pallas as pl
from jax.experimental.pallas import tpu as pltpu
from jax.experimental.pallas import tpu_sc as plsc

B = 112
ROIS = 90
RP = 96
DEG = 32
EPG = ROIS * DEG
H0, H1, H3 = 128, 256, 256
D2, D3, C = 64, 16, 2
NW = 32
AW = RP * RP


def _sc_adj_body(src_hbm, dst_hbm, w_hbm, out_hbm, a_v, src_v, dst_v, w_v):
    wid = lax.axis_index("s") * 2 + lax.axis_index("c")
    for t in range(4):
        g = t * NW + wid

        @pl.when(g < B)
        def _():
            def zero_body(i, _):
                a_v[pl.ds(i * 16, 16)] = jnp.zeros((16,), jnp.float32)
                return 0

            lax.fori_loop(0, AW // 16, zero_body, 0)

            eoff = g * EPG
            pltpu.sync_copy(src_hbm.at[pl.ds(eoff, EPG)], src_v)
            pltpu.sync_copy(dst_hbm.at[pl.ds(eoff, EPG)], dst_v)
            pltpu.sync_copy(w_hbm.at[pl.ds(eoff, EPG)], w_v)

            base = g * ROIS

            def edge_body(i, _):
                s16 = src_v[pl.ds(i * 16, 16)]
                d16 = dst_v[pl.ds(i * 16, 16)]
                w16 = w_v[pl.ds(i * 16, 16)]
                idx = (d16 - base) * RP + (s16 - base)
                plsc.addupdate_scatter(a_v, [idx], w16)
                return 0

            lax.fori_loop(0, EPG // 16, edge_body, 0)

            pltpu.sync_copy(a_v, out_hbm.at[g])


def _build_adjacency(src, dst, w):
    mesh = plsc.VectorSubcoreMesh(core_axis_name="c", subcore_axis_name="s")
    kern = pl.kernel(
        _sc_adj_body,
        out_type=jax.ShapeDtypeStruct((B, AW), jnp.float32),
        mesh=mesh,
        compiler_params=pltpu.CompilerParams(needs_layout_passes=False),
        scratch_types=[
            pltpu.VMEM((AW,), jnp.float32),
            pltpu.VMEM((EPG,), jnp.int32),
            pltpu.VMEM((EPG,), jnp.int32),
            pltpu.VMEM((EPG,), jnp.float32),
        ],
    )
    return kern(src, dst, w)


GB = 8


def _conv_body(a_ref, x_ref, w1_ref, b1_ref, w3_ref, b3_ref, h1_ref, h3_ref):
    r_i = lax.broadcasted_iota(jnp.int32, (RP, RP), 0)
    c_i = lax.broadcasted_iota(jnp.int32, (RP, RP), 1)
    eye = jnp.where(r_i == c_i, 1.0, 0.0)
    node = lax.broadcasted_iota(jnp.int32, (RP, 1), 0)
    self_w = jnp.where(node < ROIS, 1.0, 0.0)

    xg = x_ref[...].reshape(GB * RP, H0)
    xw1 = jnp.dot(xg, w1_ref[...], preferred_element_type=jnp.float32)

    ans, dinvs, h1s = [], [], []
    for j in range(GB):
        A = a_ref[j]
        deg = jnp.sum(A, axis=1, keepdims=True) + self_w
        dinv = jnp.where(deg > 0, lax.rsqrt(deg), 0.0)
        An = A + eye
        u = dinv * xw1[j * RP:(j + 1) * RP]
        h1 = jax.nn.relu(
            dinv * jnp.dot(An, u, preferred_element_type=jnp.float32)
            + b1_ref[...])
        h1_ref[:, j, 0, :] = h1[:ROIS]
        ans.append(An)
        dinvs.append(dinv)
        h1s.append(h1)

    xw2 = jnp.dot(jnp.concatenate(h1s, axis=0), w3_ref[...],
                  preferred_element_type=jnp.float32)
    for j in range(GB):
        u = dinvs[j] * xw2[j * RP:(j + 1) * RP]
        h3 = jax.nn.relu(
            dinvs[j] * jnp.dot(ans[j], u, preferred_element_type=jnp.float32)
            + b3_ref[...])
        h3_ref[:, j, 0, :] = h3[:ROIS]


def _gcn_layers(A, xp, W1, b1, W3, b3):
    h_shape = jax.ShapeDtypeStruct((ROIS, B, 1, H1), jnp.float32)
    return pl.pallas_call(
        _conv_body,
        grid=(B // GB,),
        in_specs=[
            pl.BlockSpec((GB, RP, RP), lambda g: (g, 0, 0)),
            pl.BlockSpec((GB, RP, H0), lambda g: (g, 0, 0)),
            pl.BlockSpec((H0, H1), lambda g: (0, 0)),
            pl.BlockSpec((1, H1), lambda g: (0, 0)),
            pl.BlockSpec((H1, H3), lambda g: (0, 0)),
            pl.BlockSpec((1, H3), lambda g: (0, 0)),
        ],
        out_specs=[
            pl.BlockSpec((ROIS, GB, 1, H1), lambda g: (0, g, 0, 0)),
            pl.BlockSpec((ROIS, GB, 1, H3), lambda g: (0, g, 0, 0)),
        ],
        out_shape=[h_shape, h_shape],
    )(A, xp, W1, b1.reshape(1, -1), W3, b3.reshape(1, -1))


def _head_body(h1_ref, h3_ref, wt_ref, wb_ref, fc1b_ref, fc2w_ref, fc2b_ref,
               fc3w_ref, fc3b_ref, g1_ref, bb1_ref, g2_ref, bb2_ref,
               out_ref, acc):
    r = pl.program_id(0)

    @pl.when(r == 0)
    def _():
        acc[...] = jnp.zeros_like(acc)

    h1r = h1_ref[0, :, 0, :]
    h3r = h3_ref[0, :, 0, :]
    acc[...] += (jnp.dot(h1r, wt_ref[0], preferred_element_type=jnp.float32)
                 + jnp.dot(h3r, wb_ref[0], preferred_element_type=jnp.float32))

    @pl.when(r == ROIS - 1)
    def _():
        y = jax.nn.relu(acc[...] + fc1b_ref[...])
        mu = jnp.mean(y, axis=0, keepdims=True)
        var = jnp.mean((y - mu) ** 2, axis=0, keepdims=True)
        y = g1_ref[...] * (y - mu) / jnp.sqrt(var + 1e-5) + bb1_ref[...]
        y = jax.nn.relu(
            jnp.dot(y, fc2w_ref[...], preferred_element_type=jnp.float32)
            + fc2b_ref[...])
        mu2 = jnp.mean(y, axis=0, keepdims=True)
        var2 = jnp.mean((y - mu2) ** 2, axis=0, keepdims=True)
        y = g2_ref[...] * (y - mu2) / jnp.sqrt(var2 + 1e-5) + bb2_ref[...]
        z = (jnp.dot(y, fc3w_ref[...], preferred_element_type=jnp.float32)
             + fc3b_ref[...])
        m = jnp.max(z, axis=1, keepdims=True)
        lse = m + jnp.log(jnp.sum(jnp.exp(z - m), axis=1, keepdims=True))
        out_ref[...] = z - lse


def _fc_head(h1, h3, wt, wb, fc1b, fc2W, fc2b, fc3W, fc3b,
             bn1g, bn1b, bn2g, bn2b):
    return pl.pallas_call(
        _head_body,
        grid=(ROIS,),
        in_specs=[
            pl.BlockSpec((1, B, 1, H1), lambda r: (r, 0, 0, 0)),
            pl.BlockSpec((1, B, 1, H3), lambda r: (r, 0, 0, 0)),
            pl.BlockSpec((1, H1, D2), lambda r: (r, 0, 0)),
            pl.BlockSpec((1, H3, D2), lambda r: (r, 0, 0)),
            pl.BlockSpec((1, D2), lambda r: (0, 0)),
            pl.BlockSpec((D2, D3), lambda r: (0, 0)),
            pl.BlockSpec((1, D3), lambda r: (0, 0)),
            pl.BlockSpec((D3, C), lambda r: (0, 0)),
            pl.BlockSpec((1, C), lambda r: (0, 0)),
            pl.BlockSpec((1, D2), lambda r: (0, 0)),
            pl.BlockSpec((1, D2), lambda r: (0, 0)),
            pl.BlockSpec((1, D3), lambda r: (0, 0)),
            pl.BlockSpec((1, D3), lambda r: (0, 0)),
        ],
        out_specs=pl.BlockSpec((B, C), lambda r: (0, 0)),
        out_shape=jax.ShapeDtypeStruct((B, C), jnp.float32),
        scratch_shapes=[pltpu.VMEM((B, D2), jnp.float32)],
    )(h1, h3, wt, wb, fc1b.reshape(1, -1), fc2W, fc2b.reshape(1, -1),
      fc3W, fc3b.reshape(1, -1), bn1g.reshape(1, -1), bn1b.reshape(1, -1),
      bn2g.reshape(1, -1), bn2b.reshape(1, -1))


def kernel(x, edge_index, edge_weight, batch, W1, b1, W3, b3,
           fc1W, fc1b, fc2W, fc2b, fc3W, fc3b, bn1g, bn1b, bn2g, bn2b):
    src = edge_index[0]
    dst = edge_index[1]
    adj = _build_adjacency(src, dst, edge_weight)
    A = adj.reshape(B, RP, RP)
    xp = jnp.pad(x.reshape(B, ROIS, H0), ((0, 0), (0, RP - ROIS), (0, 0)))
    h1, h3 = _gcn_layers(A, xp, W1, b1, W3, b3)
    wt = fc1W[:ROIS * H1].reshape(ROIS, H1, D2)
    wb = fc1W[ROIS * H1:].reshape(ROIS, H3, D2)
    return _fc_head(h1, h3, wt, wb, fc1b, fc2W, fc2b, fc3W, fc3b,
                    bn1g, bn1b, bn2g, bn2b)


# speedup vs baseline: 67.8790x; 1.1305x over previous
import jax
import jax.numpy as jnp
from jax import lax
from jax.experimental import pallas as pl
from jax.experimental.pallas import tpu as pltpu
from jax.experimental.pallas import tpu_sc as plsc

B = 112
ROIS = 90
RP = 96
DEG = 32
EPG = ROIS * DEG
H0, H1, H3 = 128, 256, 256
D2, D3, C = 64, 16, 2
NW = 32
AW = RP * RP


def _sc_adj_body(src_hbm, dst_hbm, w_hbm, out_hbm, a_v, src_v, dst_v, w_v):
    wid = lax.axis_index("s") * 2 + lax.axis_index("c")
    for t in range(4):
        g = t * NW + wid

        @pl.when(g < B)
        def _():
            def zero_body(i, _):
                a_v[pl.ds(i * 16, 16)] = jnp.zeros((16,), jnp.float32)
                return 0

            lax.fori_loop(0, AW // 16, zero_body, 0)

            eoff = g * EPG
            pltpu.sync_copy(src_hbm.at[pl.ds(eoff, EPG)], src_v)
            pltpu.sync_copy(dst_hbm.at[pl.ds(eoff, EPG)], dst_v)
            pltpu.sync_copy(w_hbm.at[pl.ds(eoff, EPG)], w_v)

            base = g * ROIS

            def edge_body(i, _):
                s16 = src_v[pl.ds(i * 16, 16)]
                d16 = dst_v[pl.ds(i * 16, 16)]
                w16 = w_v[pl.ds(i * 16, 16)]
                idx = (d16 - base) * RP + (s16 - base)
                plsc.addupdate_scatter(a_v, [idx], w16)
                return 0

            lax.fori_loop(0, EPG // 16, edge_body, 0)

            pltpu.sync_copy(a_v, out_hbm.at[g])


def _build_adjacency(src, dst, w):
    mesh = plsc.VectorSubcoreMesh(core_axis_name="c", subcore_axis_name="s")
    kern = pl.kernel(
        _sc_adj_body,
        out_type=jax.ShapeDtypeStruct((B, AW), jnp.float32),
        mesh=mesh,
        compiler_params=pltpu.CompilerParams(needs_layout_passes=False),
        scratch_types=[
            pltpu.VMEM((AW,), jnp.float32),
            pltpu.VMEM((EPG,), jnp.int32),
            pltpu.VMEM((EPG,), jnp.int32),
            pltpu.VMEM((EPG,), jnp.float32),
        ],
    )
    return kern(src, dst, w)


GB = 8


def _conv_body(a_ref, x_ref, w1_ref, b1_ref, w3_ref, b3_ref, h1_ref, h3_ref):
    r_i = lax.broadcasted_iota(jnp.int32, (RP, RP), 0)
    c_i = lax.broadcasted_iota(jnp.int32, (RP, RP), 1)
    eye = jnp.where(r_i == c_i, 1.0, 0.0)
    node = lax.broadcasted_iota(jnp.int32, (RP, 1), 0)
    self_w = jnp.where(node < ROIS, 1.0, 0.0)

    xg = x_ref[...].reshape(GB * RP, H0)
    xw1 = jnp.dot(xg, w1_ref[...], preferred_element_type=jnp.float32)

    ans, dinvs, h1s = [], [], []
    for j in range(GB):
        A = a_ref[j]
        deg = jnp.sum(A, axis=1, keepdims=True) + self_w
        dinv = jnp.where(deg > 0, lax.rsqrt(deg), 0.0)
        An = A + eye
        u = dinv * xw1[j * RP:(j + 1) * RP]
        h1 = jax.nn.relu(
            dinv * jnp.dot(An, u, preferred_element_type=jnp.float32)
            + b1_ref[...])
        h1_ref[:, j, 0, :] = h1
        ans.append(An)
        dinvs.append(dinv)
        h1s.append(h1)

    xw2 = jnp.dot(jnp.concatenate(h1s, axis=0), w3_ref[...],
                  preferred_element_type=jnp.float32)
    for j in range(GB):
        u = dinvs[j] * xw2[j * RP:(j + 1) * RP]
        h3 = jax.nn.relu(
            dinvs[j] * jnp.dot(ans[j], u, preferred_element_type=jnp.float32)
            + b3_ref[...])
        h3_ref[:, j, 0, :] = h3


def _gcn_layers(A, xp, W1, b1, W3, b3):
    h_shape = jax.ShapeDtypeStruct((RP, B, 1, H1), jnp.float32)
    return pl.pallas_call(
        _conv_body,
        grid=(B // GB,),
        in_specs=[
            pl.BlockSpec((GB, RP, RP), lambda g: (g, 0, 0)),
            pl.BlockSpec((GB, RP, H0), lambda g: (g, 0, 0)),
            pl.BlockSpec((H0, H1), lambda g: (0, 0)),
            pl.BlockSpec((1, H1), lambda g: (0, 0)),
            pl.BlockSpec((H1, H3), lambda g: (0, 0)),
            pl.BlockSpec((1, H3), lambda g: (0, 0)),
        ],
        out_specs=[
            pl.BlockSpec((RP, GB, 1, H1), lambda g: (0, g, 0, 0)),
            pl.BlockSpec((RP, GB, 1, H3), lambda g: (0, g, 0, 0)),
        ],
        out_shape=[h_shape, h_shape],
    )(A, xp, W1, b1.reshape(1, -1), W3, b3.reshape(1, -1))


RC = 8


def _head_body(h1_ref, h3_ref, wt_ref, wb_ref, fc1b_ref, fc2w_ref, fc2b_ref,
               fc3w_ref, fc3b_ref, g1_ref, bb1_ref, g2_ref, bb2_ref,
               out_ref, acc):
    r = pl.program_id(0)

    @pl.when(r == 0)
    def _():
        acc[...] = jnp.zeros_like(acc)

    part = jnp.zeros((B, D2), jnp.float32)
    for j in range(RC):
        h1r = h1_ref[j, :, 0, :]
        h3r = h3_ref[j, :, 0, :]
        part += (jnp.dot(h1r, wt_ref[j], preferred_element_type=jnp.float32)
                 + jnp.dot(h3r, wb_ref[j], preferred_element_type=jnp.float32))
    acc[...] += part

    @pl.when(r == RP // RC - 1)
    def _():
        y = jax.nn.relu(acc[...] + fc1b_ref[...])
        mu = jnp.mean(y, axis=0, keepdims=True)
        var = jnp.mean((y - mu) ** 2, axis=0, keepdims=True)
        y = g1_ref[...] * (y - mu) / jnp.sqrt(var + 1e-5) + bb1_ref[...]
        y = jax.nn.relu(
            jnp.dot(y, fc2w_ref[...], preferred_element_type=jnp.float32)
            + fc2b_ref[...])
        mu2 = jnp.mean(y, axis=0, keepdims=True)
        var2 = jnp.mean((y - mu2) ** 2, axis=0, keepdims=True)
        y = g2_ref[...] * (y - mu2) / jnp.sqrt(var2 + 1e-5) + bb2_ref[...]
        z = (jnp.dot(y, fc3w_ref[...], preferred_element_type=jnp.float32)
             + fc3b_ref[...])
        m = jnp.max(z, axis=1, keepdims=True)
        lse = m + jnp.log(jnp.sum(jnp.exp(z - m), axis=1, keepdims=True))
        out_ref[...] = z - lse


def _fc_head(h1, h3, wt, wb, fc1b, fc2W, fc2b, fc3W, fc3b,
             bn1g, bn1b, bn2g, bn2b):
    return pl.pallas_call(
        _head_body,
        grid=(RP // RC,),
        in_specs=[
            pl.BlockSpec((RC, B, 1, H1), lambda r: (r, 0, 0, 0)),
            pl.BlockSpec((RC, B, 1, H3), lambda r: (r, 0, 0, 0)),
            pl.BlockSpec((RC, H1, D2), lambda r: (r, 0, 0)),
            pl.BlockSpec((RC, H3, D2), lambda r: (r, 0, 0)),
            pl.BlockSpec((1, D2), lambda r: (0, 0)),
            pl.BlockSpec((D2, D3), lambda r: (0, 0)),
            pl.BlockSpec((1, D3), lambda r: (0, 0)),
            pl.BlockSpec((D3, C), lambda r: (0, 0)),
            pl.BlockSpec((1, C), lambda r: (0, 0)),
            pl.BlockSpec((1, D2), lambda r: (0, 0)),
            pl.BlockSpec((1, D2), lambda r: (0, 0)),
            pl.BlockSpec((1, D3), lambda r: (0, 0)),
            pl.BlockSpec((1, D3), lambda r: (0, 0)),
        ],
        out_specs=pl.BlockSpec((B, C), lambda r: (0, 0)),
        out_shape=jax.ShapeDtypeStruct((B, C), jnp.float32),
        scratch_shapes=[pltpu.VMEM((B, D2), jnp.float32)],
    )(h1, h3, wt, wb, fc1b.reshape(1, -1), fc2W, fc2b.reshape(1, -1),
      fc3W, fc3b.reshape(1, -1), bn1g.reshape(1, -1), bn1b.reshape(1, -1),
      bn2g.reshape(1, -1), bn2b.reshape(1, -1))


def kernel(x, edge_index, edge_weight, batch, W1, b1, W3, b3,
           fc1W, fc1b, fc2W, fc2b, fc3W, fc3b, bn1g, bn1b, bn2g, bn2b):
    src = edge_index[0]
    dst = edge_index[1]
    adj = _build_adjacency(src, dst, edge_weight)
    A = adj.reshape(B, RP, RP)
    xp = jnp.pad(x.reshape(B, ROIS, H0), ((0, 0), (0, RP - ROIS), (0, 0)))
    h1, h3 = _gcn_layers(A, xp, W1, b1, W3, b3)
    wt = jnp.pad(fc1W[:ROIS * H1].reshape(ROIS, H1, D2),
                 ((0, RP - ROIS), (0, 0), (0, 0)))
    wb = jnp.pad(fc1W[ROIS * H1:].reshape(ROIS, H3, D2),
                 ((0, RP - ROIS), (0, 0), (0, 0)))
    return _fc_head(h1, h3, wt, wb, fc1b, fc2W, fc2b, fc3W, fc3b,
                    bn1g, bn1b, bn2g, bn2b)


# speedup vs baseline: 71.7930x; 1.0577x over previous
import jax
import jax.numpy as jnp
from jax import lax
from jax.experimental import pallas as pl
from jax.experimental.pallas import tpu as pltpu
from jax.experimental.pallas import tpu_sc as plsc

B = 112
ROIS = 90
RP = 96
DEG = 32
EPG = ROIS * DEG
H0, H1, H3 = 128, 256, 256
D2, D3, C = 64, 16, 2
NW = 32
AW = RP * RP


def _sc_adj_body(src_hbm, dst_hbm, w_hbm, out_hbm, a_v, src_v, dst_v, w_v):
    wid = lax.axis_index("s") * 2 + lax.axis_index("c")
    for t in range(4):
        g = t * NW + wid

        @pl.when(g < B)
        def _():
            def zero_body(i, _):
                a_v[pl.ds(i * 16, 16)] = jnp.zeros((16,), jnp.float32)
                return 0

            lax.fori_loop(0, AW // 16, zero_body, 0)

            eoff = g * EPG
            pltpu.sync_copy(src_hbm.at[pl.ds(eoff, EPG)], src_v)
            pltpu.sync_copy(dst_hbm.at[pl.ds(eoff, EPG)], dst_v)
            pltpu.sync_copy(w_hbm.at[pl.ds(eoff, EPG)], w_v)

            base = g * ROIS

            def edge_body(i, _):
                s16 = src_v[pl.ds(i * 16, 16)]
                d16 = dst_v[pl.ds(i * 16, 16)]
                w16 = w_v[pl.ds(i * 16, 16)]
                idx = (d16 - base) * RP + (s16 - base)
                plsc.addupdate_scatter(a_v, [idx], w16)
                return 0

            lax.fori_loop(0, EPG // 16, edge_body, 0)

            pltpu.sync_copy(a_v, out_hbm.at[g])


def _build_adjacency(src, dst, w):
    mesh = plsc.VectorSubcoreMesh(core_axis_name="c", subcore_axis_name="s")
    kern = pl.kernel(
        _sc_adj_body,
        out_type=jax.ShapeDtypeStruct((B, AW), jnp.float32),
        mesh=mesh,
        compiler_params=pltpu.CompilerParams(needs_layout_passes=False),
        scratch_types=[
            pltpu.VMEM((AW,), jnp.float32),
            pltpu.VMEM((EPG,), jnp.int32),
            pltpu.VMEM((EPG,), jnp.int32),
            pltpu.VMEM((EPG,), jnp.float32),
        ],
    )
    return kern(src, dst, w)


GB = 8


def _conv_body(a_ref, x_ref, w1_ref, b1_ref, w3_ref, b3_ref, h1_ref, h3_ref):
    r_i = lax.broadcasted_iota(jnp.int32, (RP, RP), 0)
    c_i = lax.broadcasted_iota(jnp.int32, (RP, RP), 1)
    eye = jnp.where(r_i == c_i, 1.0, 0.0)
    node = lax.broadcasted_iota(jnp.int32, (RP, 1), 0)
    self_w = jnp.where(node < ROIS, 1.0, 0.0)

    bf = jnp.bfloat16
    xg = x_ref[...].reshape(GB * RP, H0)
    xw1 = jnp.dot(xg.astype(bf), w1_ref[...],
                  preferred_element_type=jnp.float32)

    ans, dinvs, h1s = [], [], []
    for j in range(GB):
        A = a_ref[j]
        deg = jnp.sum(A, axis=1, keepdims=True) + self_w
        dinv = jnp.where(deg > 0, lax.rsqrt(deg), 0.0)
        An = (A + eye).astype(bf)
        u = (dinv * xw1[j * RP:(j + 1) * RP]).astype(bf)
        h1 = jax.nn.relu(
            dinv * jnp.dot(An, u, preferred_element_type=jnp.float32)
            + b1_ref[...]).astype(bf)
        h1_ref[:, j, 0, :] = h1
        ans.append(An)
        dinvs.append(dinv)
        h1s.append(h1)

    xw2 = jnp.dot(jnp.concatenate(h1s, axis=0), w3_ref[...],
                  preferred_element_type=jnp.float32)
    for j in range(GB):
        u = (dinvs[j] * xw2[j * RP:(j + 1) * RP]).astype(bf)
        h3 = jax.nn.relu(
            dinvs[j] * jnp.dot(ans[j], u, preferred_element_type=jnp.float32)
            + b3_ref[...]).astype(bf)
        h3_ref[:, j, 0, :] = h3


def _gcn_layers(A, xp, W1, b1, W3, b3):
    h_shape = jax.ShapeDtypeStruct((RP, B, 1, H1), jnp.bfloat16)
    return pl.pallas_call(
        _conv_body,
        grid=(B // GB,),
        in_specs=[
            pl.BlockSpec((GB, RP, RP), lambda g: (g, 0, 0)),
            pl.BlockSpec((GB, RP, H0), lambda g: (g, 0, 0)),
            pl.BlockSpec((H0, H1), lambda g: (0, 0)),
            pl.BlockSpec((1, H1), lambda g: (0, 0)),
            pl.BlockSpec((H1, H3), lambda g: (0, 0)),
            pl.BlockSpec((1, H3), lambda g: (0, 0)),
        ],
        out_specs=[
            pl.BlockSpec((RP, GB, 1, H1), lambda g: (0, g, 0, 0)),
            pl.BlockSpec((RP, GB, 1, H3), lambda g: (0, g, 0, 0)),
        ],
        out_shape=[h_shape, h_shape],
    )(A, xp, W1.astype(jnp.bfloat16), b1.reshape(1, -1),
      W3.astype(jnp.bfloat16), b3.reshape(1, -1))


RC = 8


def _head_body(h1_ref, h3_ref, wt_ref, wb_ref, fc1b_ref, fc2w_ref, fc2b_ref,
               fc3w_ref, fc3b_ref, g1_ref, bb1_ref, g2_ref, bb2_ref,
               out_ref, acc):
    r = pl.program_id(0)

    @pl.when(r == 0)
    def _():
        acc[...] = jnp.zeros_like(acc)

    part = jnp.zeros((B, D2), jnp.float32)
    for j in range(RC):
        h1r = h1_ref[j, :, 0, :]
        h3r = h3_ref[j, :, 0, :]
        part += (jnp.dot(h1r, wt_ref[j], preferred_element_type=jnp.float32)
                 + jnp.dot(h3r, wb_ref[j], preferred_element_type=jnp.float32))
    acc[...] += part

    @pl.when(r == RP // RC - 1)
    def _():
        y = jax.nn.relu(acc[...] + fc1b_ref[...])
        mu = jnp.mean(y, axis=0, keepdims=True)
        var = jnp.mean((y - mu) ** 2, axis=0, keepdims=True)
        y = g1_ref[...] * (y - mu) / jnp.sqrt(var + 1e-5) + bb1_ref[...]
        y = jax.nn.relu(
            jnp.dot(y, fc2w_ref[...], preferred_element_type=jnp.float32)
            + fc2b_ref[...])
        mu2 = jnp.mean(y, axis=0, keepdims=True)
        var2 = jnp.mean((y - mu2) ** 2, axis=0, keepdims=True)
        y = g2_ref[...] * (y - mu2) / jnp.sqrt(var2 + 1e-5) + bb2_ref[...]
        z = (jnp.dot(y, fc3w_ref[...], preferred_element_type=jnp.float32)
             + fc3b_ref[...])
        m = jnp.max(z, axis=1, keepdims=True)
        lse = m + jnp.log(jnp.sum(jnp.exp(z - m), axis=1, keepdims=True))
        out_ref[...] = z - lse


def _fc_head(h1, h3, wt, wb, fc1b, fc2W, fc2b, fc3W, fc3b,
             bn1g, bn1b, bn2g, bn2b):
    return pl.pallas_call(
        _head_body,
        grid=(RP // RC,),
        in_specs=[
            pl.BlockSpec((RC, B, 1, H1), lambda r: (r, 0, 0, 0)),
            pl.BlockSpec((RC, B, 1, H3), lambda r: (r, 0, 0, 0)),
            pl.BlockSpec((RC, H1, D2), lambda r: (r, 0, 0)),
            pl.BlockSpec((RC, H3, D2), lambda r: (r, 0, 0)),
            pl.BlockSpec((1, D2), lambda r: (0, 0)),
            pl.BlockSpec((D2, D3), lambda r: (0, 0)),
            pl.BlockSpec((1, D3), lambda r: (0, 0)),
            pl.BlockSpec((D3, C), lambda r: (0, 0)),
            pl.BlockSpec((1, C), lambda r: (0, 0)),
            pl.BlockSpec((1, D2), lambda r: (0, 0)),
            pl.BlockSpec((1, D2), lambda r: (0, 0)),
            pl.BlockSpec((1, D3), lambda r: (0, 0)),
            pl.BlockSpec((1, D3), lambda r: (0, 0)),
        ],
        out_specs=pl.BlockSpec((B, C), lambda r: (0, 0)),
        out_shape=jax.ShapeDtypeStruct((B, C), jnp.float32),
        scratch_shapes=[pltpu.VMEM((B, D2), jnp.float32)],
    )(h1, h3, wt, wb, fc1b.reshape(1, -1), fc2W, fc2b.reshape(1, -1),
      fc3W, fc3b.reshape(1, -1), bn1g.reshape(1, -1), bn1b.reshape(1, -1),
      bn2g.reshape(1, -1), bn2b.reshape(1, -1))


def kernel(x, edge_index, edge_weight, batch, W1, b1, W3, b3,
           fc1W, fc1b, fc2W, fc2b, fc3W, fc3b, bn1g, bn1b, bn2g, bn2b):
    src = edge_index[0]
    dst = edge_index[1]
    adj = _build_adjacency(src, dst, edge_weight)
    A = adj.reshape(B, RP, RP)
    xp = jnp.pad(x.reshape(B, ROIS, H0), ((0, 0), (0, RP - ROIS), (0, 0)))
    h1, h3 = _gcn_layers(A, xp, W1, b1, W3, b3)
    wt = jnp.pad(fc1W[:ROIS * H1].reshape(ROIS, H1, D2),
                 ((0, RP - ROIS), (0, 0), (0, 0))).astype(jnp.bfloat16)
    wb = jnp.pad(fc1W[ROIS * H1:].reshape(ROIS, H3, D2),
                 ((0, RP - ROIS), (0, 0), (0, 0))).astype(jnp.bfloat16)
    return _fc_head(h1, h3, wt, wb, fc1b, fc2W, fc2b, fc3W, fc3b,
                    bn1g, bn1b, bn2g, bn2b)


# speedup vs baseline: 72.3367x; 1.0076x over previous
import jax
import jax.numpy as jnp
from jax import lax
from jax.experimental import pallas as pl
from jax.experimental.pallas import tpu as pltpu
from jax.experimental.pallas import tpu_sc as plsc

B = 112
ROIS = 90
RP = 96
DEG = 32
EPG = ROIS * DEG
H0, H1, H3 = 128, 256, 256
D2, D3, C = 64, 16, 2
NW = 32
AW = RP * RP


def _sc_adj_body(src_hbm, dst_hbm, w_hbm, out_hbm, a_v, src_v, dst_v, w_v,
                 sem_s, sem_d, sem_w):
    wid = lax.axis_index("s") * 2 + lax.axis_index("c")
    zeros16 = jnp.zeros((16,), jnp.float32)
    for t in range(4):
        g = t * NW + wid

        @pl.when(g < B)
        def _():
            eoff = g * EPG
            cp_s = pltpu.async_copy(src_hbm.at[pl.ds(eoff, EPG)], src_v, sem_s)
            cp_d = pltpu.async_copy(dst_hbm.at[pl.ds(eoff, EPG)], dst_v, sem_d)
            cp_w = pltpu.async_copy(w_hbm.at[pl.ds(eoff, EPG)], w_v, sem_w)

            def zero_body(i, _):
                b0 = i * 256
                for k in range(16):
                    a_v[pl.ds(b0 + k * 16, 16)] = zeros16
                return 0

            lax.fori_loop(0, AW // 256, zero_body, 0)
            cp_s.wait()
            cp_d.wait()
            cp_w.wait()

            base97 = g * ROIS * (RP + 1)

            def edge_body(i, _):
                b0 = i * 64
                for k in range(4):
                    off = b0 + k * 16
                    s16 = src_v[pl.ds(off, 16)]
                    d16 = dst_v[pl.ds(off, 16)]
                    w16 = w_v[pl.ds(off, 16)]
                    idx = d16 * RP + s16 - base97
                    plsc.addupdate_scatter(a_v, [idx], w16)
                return 0

            lax.fori_loop(0, EPG // 64, edge_body, 0)

            pltpu.sync_copy(a_v, out_hbm.at[g])


def _build_adjacency(src, dst, w):
    mesh = plsc.VectorSubcoreMesh(core_axis_name="c", subcore_axis_name="s")
    kern = pl.kernel(
        _sc_adj_body,
        out_type=jax.ShapeDtypeStruct((B, AW), jnp.float32),
        mesh=mesh,
        compiler_params=pltpu.CompilerParams(needs_layout_passes=False),
        scratch_types=[
            pltpu.VMEM((AW,), jnp.float32),
            pltpu.VMEM((EPG,), jnp.int32),
            pltpu.VMEM((EPG,), jnp.int32),
            pltpu.VMEM((EPG,), jnp.float32),
            pltpu.SemaphoreType.DMA,
            pltpu.SemaphoreType.DMA,
            pltpu.SemaphoreType.DMA,
        ],
    )
    return kern(src, dst, w)


GB = 8


def _conv_body(a_ref, x_ref, w1_ref, b1_ref, w3_ref, b3_ref, h1_ref, h3_ref):
    r_i = lax.broadcasted_iota(jnp.int32, (RP, RP), 0)
    c_i = lax.broadcasted_iota(jnp.int32, (RP, RP), 1)
    eye = jnp.where(r_i == c_i, 1.0, 0.0)
    node = lax.broadcasted_iota(jnp.int32, (RP, 1), 0)
    self_w = jnp.where(node < ROIS, 1.0, 0.0)

    bf = jnp.bfloat16
    xg = x_ref[...].reshape(GB * RP, H0)
    xw1 = jnp.dot(xg.astype(bf), w1_ref[...],
                  preferred_element_type=jnp.float32)

    ans, dinvs, h1s = [], [], []
    for j in range(GB):
        A = a_ref[j]
        deg = jnp.sum(A, axis=1, keepdims=True) + self_w
        dinv = jnp.where(deg > 0, lax.rsqrt(deg), 0.0)
        An = (A + eye).astype(bf)
        u = (dinv * xw1[j * RP:(j + 1) * RP]).astype(bf)
        h1 = jax.nn.relu(
            dinv * jnp.dot(An, u, preferred_element_type=jnp.float32)
            + b1_ref[...]).astype(bf)
        h1_ref[:, j, 0, :] = h1
        ans.append(An)
        dinvs.append(dinv)
        h1s.append(h1)

    xw2 = jnp.dot(jnp.concatenate(h1s, axis=0), w3_ref[...],
                  preferred_element_type=jnp.float32)
    for j in range(GB):
        u = (dinvs[j] * xw2[j * RP:(j + 1) * RP]).astype(bf)
        h3 = jax.nn.relu(
            dinvs[j] * jnp.dot(ans[j], u, preferred_element_type=jnp.float32)
            + b3_ref[...]).astype(bf)
        h3_ref[:, j, 0, :] = h3


def _gcn_layers(A, xp, W1, b1, W3, b3):
    h_shape = jax.ShapeDtypeStruct((RP, B, 1, H1), jnp.bfloat16)
    return pl.pallas_call(
        _conv_body,
        grid=(B // GB,),
        in_specs=[
            pl.BlockSpec((GB, RP, RP), lambda g: (g, 0, 0)),
            pl.BlockSpec((GB, RP, H0), lambda g: (g, 0, 0)),
            pl.BlockSpec((H0, H1), lambda g: (0, 0)),
            pl.BlockSpec((1, H1), lambda g: (0, 0)),
            pl.BlockSpec((H1, H3), lambda g: (0, 0)),
            pl.BlockSpec((1, H3), lambda g: (0, 0)),
        ],
        out_specs=[
            pl.BlockSpec((RP, GB, 1, H1), lambda g: (0, g, 0, 0)),
            pl.BlockSpec((RP, GB, 1, H3), lambda g: (0, g, 0, 0)),
        ],
        out_shape=[h_shape, h_shape],
    )(A, xp, W1.astype(jnp.bfloat16), b1.reshape(1, -1),
      W3.astype(jnp.bfloat16), b3.reshape(1, -1))


RC = 8


def _head_body(h1_ref, h3_ref, wt_ref, wb_ref, fc1b_ref, fc2w_ref, fc2b_ref,
               fc3w_ref, fc3b_ref, g1_ref, bb1_ref, g2_ref, bb2_ref,
               out_ref, acc):
    r = pl.program_id(0)

    @pl.when(r == 0)
    def _():
        acc[...] = jnp.zeros_like(acc)

    part = jnp.zeros((B, D2), jnp.float32)
    for j in range(RC):
        h1r = h1_ref[j, :, 0, :]
        h3r = h3_ref[j, :, 0, :]
        part += (jnp.dot(h1r, wt_ref[j], preferred_element_type=jnp.float32)
                 + jnp.dot(h3r, wb_ref[j], preferred_element_type=jnp.float32))
    acc[...] += part

    @pl.when(r == RP // RC - 1)
    def _():
        y = jax.nn.relu(acc[...] + fc1b_ref[...])
        mu = jnp.mean(y, axis=0, keepdims=True)
        var = jnp.mean((y - mu) ** 2, axis=0, keepdims=True)
        y = g1_ref[...] * (y - mu) / jnp.sqrt(var + 1e-5) + bb1_ref[...]
        y = jax.nn.relu(
            jnp.dot(y, fc2w_ref[...], preferred_element_type=jnp.float32)
            + fc2b_ref[...])
        mu2 = jnp.mean(y, axis=0, keepdims=True)
        var2 = jnp.mean((y - mu2) ** 2, axis=0, keepdims=True)
        y = g2_ref[...] * (y - mu2) / jnp.sqrt(var2 + 1e-5) + bb2_ref[...]
        z = (jnp.dot(y, fc3w_ref[...], preferred_element_type=jnp.float32)
             + fc3b_ref[...])
        m = jnp.max(z, axis=1, keepdims=True)
        lse = m + jnp.log(jnp.sum(jnp.exp(z - m), axis=1, keepdims=True))
        out_ref[...] = z - lse


def _fc_head(h1, h3, wt, wb, fc1b, fc2W, fc2b, fc3W, fc3b,
             bn1g, bn1b, bn2g, bn2b):
    return pl.pallas_call(
        _head_body,
        grid=(RP // RC,),
        in_specs=[
            pl.BlockSpec((RC, B, 1, H1), lambda r: (r, 0, 0, 0)),
            pl.BlockSpec((RC, B, 1, H3), lambda r: (r, 0, 0, 0)),
            pl.BlockSpec((RC, H1, D2), lambda r: (r, 0, 0)),
            pl.BlockSpec((RC, H3, D2), lambda r: (r, 0, 0)),
            pl.BlockSpec((1, D2), lambda r: (0, 0)),
            pl.BlockSpec((D2, D3), lambda r: (0, 0)),
            pl.BlockSpec((1, D3), lambda r: (0, 0)),
            pl.BlockSpec((D3, C), lambda r: (0, 0)),
            pl.BlockSpec((1, C), lambda r: (0, 0)),
            pl.BlockSpec((1, D2), lambda r: (0, 0)),
            pl.BlockSpec((1, D2), lambda r: (0, 0)),
            pl.BlockSpec((1, D3), lambda r: (0, 0)),
            pl.BlockSpec((1, D3), lambda r: (0, 0)),
        ],
        out_specs=pl.BlockSpec((B, C), lambda r: (0, 0)),
        out_shape=jax.ShapeDtypeStruct((B, C), jnp.float32),
        scratch_shapes=[pltpu.VMEM((B, D2), jnp.float32)],
    )(h1, h3, wt, wb, fc1b.reshape(1, -1), fc2W, fc2b.reshape(1, -1),
      fc3W, fc3b.reshape(1, -1), bn1g.reshape(1, -1), bn1b.reshape(1, -1),
      bn2g.reshape(1, -1), bn2b.reshape(1, -1))


def kernel(x, edge_index, edge_weight, batch, W1, b1, W3, b3,
           fc1W, fc1b, fc2W, fc2b, fc3W, fc3b, bn1g, bn1b, bn2g, bn2b):
    src = edge_index[0]
    dst = edge_index[1]
    adj = _build_adjacency(src, dst, edge_weight)
    A = adj.reshape(B, RP, RP)
    xp = jnp.pad(x.reshape(B, ROIS, H0), ((0, 0), (0, RP - ROIS), (0, 0)))
    h1, h3 = _gcn_layers(A, xp, W1, b1, W3, b3)
    wt = jnp.pad(fc1W[:ROIS * H1].reshape(ROIS, H1, D2),
                 ((0, RP - ROIS), (0, 0), (0, 0))).astype(jnp.bfloat16)
    wb = jnp.pad(fc1W[ROIS * H1:].reshape(ROIS, H3, D2),
                 ((0, RP - ROIS), (0, 0), (0, 0))).astype(jnp.bfloat16)
    return _fc_head(h1, h3, wt, wb, fc1b, fc2W, fc2b, fc3W, fc3b,
                    bn1g, bn1b, bn2g, bn2b)


# speedup vs baseline: 86.1900x; 1.1915x over previous
import jax
import jax.numpy as jnp
from jax import lax
from jax.experimental import pallas as pl
from jax.experimental.pallas import tpu as pltpu
from jax.experimental.pallas import tpu_sc as plsc

B = 112
ROIS = 90
RP = 96
DEG = 32
EPG = ROIS * DEG
H0, H1, H3 = 128, 256, 256
D2, D3, C = 64, 16, 2
NW = 32
AW = RP * RP


def _sc_adj_body(src_hbm, dst_hbm, w_hbm, out_hbm, a_v, src_v, dst_v, w_v,
                 sem_s, sem_d, sem_w):
    wid = lax.axis_index("s") * 2 + lax.axis_index("c")
    zeros16 = jnp.zeros((16,), jnp.float32)
    for t in range(4):
        g = t * NW + wid

        @pl.when(g < B)
        def _():
            eoff = g * EPG
            cp_s = pltpu.async_copy(src_hbm.at[pl.ds(eoff, EPG)], src_v, sem_s)
            cp_d = pltpu.async_copy(dst_hbm.at[pl.ds(eoff, EPG)], dst_v, sem_d)
            cp_w = pltpu.async_copy(w_hbm.at[pl.ds(eoff, EPG)], w_v, sem_w)

            def zero_body(i, _):
                b0 = i * 256
                for k in range(16):
                    a_v[pl.ds(b0 + k * 16, 16)] = zeros16
                return 0

            lax.fori_loop(0, AW // 256, zero_body, 0)
            cp_s.wait()
            cp_d.wait()
            cp_w.wait()

            base97 = g * ROIS * (RP + 1)

            def edge_body(i, _):
                b0 = i * 64
                for k in range(4):
                    off = b0 + k * 16
                    s16 = src_v[pl.ds(off, 16)]
                    d16 = dst_v[pl.ds(off, 16)]
                    w16 = w_v[pl.ds(off, 16)]
                    idx = d16 * RP + s16 - base97
                    plsc.addupdate_scatter(a_v, [idx], w16)
                return 0

            lax.fori_loop(0, EPG // 64, edge_body, 0)

            pltpu.sync_copy(a_v, out_hbm.at[g])


def _build_adjacency(src, dst, w):
    mesh = plsc.VectorSubcoreMesh(core_axis_name="c", subcore_axis_name="s")
    kern = pl.kernel(
        _sc_adj_body,
        out_type=jax.ShapeDtypeStruct((B, AW), jnp.float32),
        mesh=mesh,
        compiler_params=pltpu.CompilerParams(needs_layout_passes=False),
        scratch_types=[
            pltpu.VMEM((AW,), jnp.float32),
            pltpu.VMEM((EPG,), jnp.int32),
            pltpu.VMEM((EPG,), jnp.int32),
            pltpu.VMEM((EPG,), jnp.float32),
            pltpu.SemaphoreType.DMA,
            pltpu.SemaphoreType.DMA,
            pltpu.SemaphoreType.DMA,
        ],
    )
    return kern(src, dst, w)


GB = 8


def _conv_body(a_ref, x_ref, w1_ref, b1_ref, w3_ref, b3_ref, h1_ref, h3_ref):
    r_i = lax.broadcasted_iota(jnp.int32, (ROIS, ROIS), 0)
    c_i = lax.broadcasted_iota(jnp.int32, (ROIS, ROIS), 1)
    eye = jnp.where(r_i == c_i, 1.0, 0.0)

    bf = jnp.bfloat16
    xg = x_ref[...].reshape(GB * ROIS, H0)
    xw1 = jnp.dot(xg.astype(bf), w1_ref[...],
                  preferred_element_type=jnp.float32)

    ans, dinvs, h1s = [], [], []
    for j in range(GB):
        A = a_ref[j, :ROIS, :ROIS]
        deg = jnp.sum(A, axis=1, keepdims=True) + 1.0
        dinv = jnp.where(deg > 0, lax.rsqrt(deg), 0.0)
        An = (A + eye).astype(bf)
        u = (dinv * xw1[j * ROIS:(j + 1) * ROIS]).astype(bf)
        h1 = jax.nn.relu(
            dinv * jnp.dot(An, u, preferred_element_type=jnp.float32)
            + b1_ref[...]).astype(bf)
        h1_ref[:, j, 0, :] = h1
        ans.append(An)
        dinvs.append(dinv)
        h1s.append(h1)

    xw2 = jnp.dot(jnp.concatenate(h1s, axis=0), w3_ref[...],
                  preferred_element_type=jnp.float32)
    for j in range(GB):
        u = (dinvs[j] * xw2[j * ROIS:(j + 1) * ROIS]).astype(bf)
        h3 = jax.nn.relu(
            dinvs[j] * jnp.dot(ans[j], u, preferred_element_type=jnp.float32)
            + b3_ref[...]).astype(bf)
        h3_ref[:, j, 0, :] = h3


def _gcn_layers(A, x3, W1, b1, W3, b3):
    h_shape = jax.ShapeDtypeStruct((ROIS, B, 1, H1), jnp.bfloat16)
    return pl.pallas_call(
        _conv_body,
        grid=(B // GB,),
        in_specs=[
            pl.BlockSpec((GB, RP, RP), lambda g: (g, 0, 0)),
            pl.BlockSpec((GB, ROIS, H0), lambda g: (g, 0, 0)),
            pl.BlockSpec((H0, H1), lambda g: (0, 0)),
            pl.BlockSpec((1, H1), lambda g: (0, 0)),
            pl.BlockSpec((H1, H3), lambda g: (0, 0)),
            pl.BlockSpec((1, H3), lambda g: (0, 0)),
        ],
        out_specs=[
            pl.BlockSpec((ROIS, GB, 1, H1), lambda g: (0, g, 0, 0)),
            pl.BlockSpec((ROIS, GB, 1, H3), lambda g: (0, g, 0, 0)),
        ],
        out_shape=[h_shape, h_shape],
    )(A, x3, W1.astype(jnp.bfloat16), b1.reshape(1, -1),
      W3.astype(jnp.bfloat16), b3.reshape(1, -1))


RC = 6


def _head_body(h1_ref, h3_ref, wt_ref, wb_ref, fc1b_ref, fc2w_ref, fc2b_ref,
               fc3w_ref, fc3b_ref, g1_ref, bb1_ref, g2_ref, bb2_ref,
               out_ref, acc):
    r = pl.program_id(0)

    @pl.when(r == 0)
    def _():
        acc[...] = jnp.zeros_like(acc)

    bf = jnp.bfloat16
    wtb = wt_ref[...].astype(bf)
    wbb = wb_ref[...].astype(bf)
    part = jnp.zeros((B, D2), jnp.float32)
    for j in range(RC):
        h1r = h1_ref[j, :, 0, :]
        h3r = h3_ref[j, :, 0, :]
        part += (jnp.dot(h1r, wtb[j], preferred_element_type=jnp.float32)
                 + jnp.dot(h3r, wbb[j], preferred_element_type=jnp.float32))
    acc[...] += part

    @pl.when(r == ROIS // RC - 1)
    def _():
        y = jax.nn.relu(acc[...] + fc1b_ref[...])
        mu = jnp.mean(y, axis=0, keepdims=True)
        var = jnp.mean((y - mu) ** 2, axis=0, keepdims=True)
        y = g1_ref[...] * (y - mu) / jnp.sqrt(var + 1e-5) + bb1_ref[...]
        y = jax.nn.relu(
            jnp.dot(y, fc2w_ref[...], preferred_element_type=jnp.float32)
            + fc2b_ref[...])
        mu2 = jnp.mean(y, axis=0, keepdims=True)
        var2 = jnp.mean((y - mu2) ** 2, axis=0, keepdims=True)
        y = g2_ref[...] * (y - mu2) / jnp.sqrt(var2 + 1e-5) + bb2_ref[...]
        z = (jnp.dot(y, fc3w_ref[...], preferred_element_type=jnp.float32)
             + fc3b_ref[...])
        m = jnp.max(z, axis=1, keepdims=True)
        lse = m + jnp.log(jnp.sum(jnp.exp(z - m), axis=1, keepdims=True))
        out_ref[...] = z - lse


def _fc_head(h1, h3, wt, wb, fc1b, fc2W, fc2b, fc3W, fc3b,
             bn1g, bn1b, bn2g, bn2b):
    return pl.pallas_call(
        _head_body,
        grid=(ROIS // RC,),
        in_specs=[
            pl.BlockSpec((RC, B, 1, H1), lambda r: (r, 0, 0, 0)),
            pl.BlockSpec((RC, B, 1, H3), lambda r: (r, 0, 0, 0)),
            pl.BlockSpec((RC, H1, D2), lambda r: (r, 0, 0)),
            pl.BlockSpec((RC, H3, D2), lambda r: (r + ROIS // RC, 0, 0)),
            pl.BlockSpec((1, D2), lambda r: (0, 0)),
            pl.BlockSpec((D2, D3), lambda r: (0, 0)),
            pl.BlockSpec((1, D3), lambda r: (0, 0)),
            pl.BlockSpec((D3, C), lambda r: (0, 0)),
            pl.BlockSpec((1, C), lambda r: (0, 0)),
            pl.BlockSpec((1, D2), lambda r: (0, 0)),
            pl.BlockSpec((1, D2), lambda r: (0, 0)),
            pl.BlockSpec((1, D3), lambda r: (0, 0)),
            pl.BlockSpec((1, D3), lambda r: (0, 0)),
        ],
        out_specs=pl.BlockSpec((B, C), lambda r: (0, 0)),
        out_shape=jax.ShapeDtypeStruct((B, C), jnp.float32),
        scratch_shapes=[pltpu.VMEM((B, D2), jnp.float32)],
    )(h1, h3, wt, wb, fc1b.reshape(1, -1), fc2W, fc2b.reshape(1, -1),
      fc3W, fc3b.reshape(1, -1), bn1g.reshape(1, -1), bn1b.reshape(1, -1),
      bn2g.reshape(1, -1), bn2b.reshape(1, -1))


def kernel(x, edge_index, edge_weight, batch, W1, b1, W3, b3,
           fc1W, fc1b, fc2W, fc2b, fc3W, fc3b, bn1g, bn1b, bn2g, bn2b):
    src = edge_index[0]
    dst = edge_index[1]
    adj = _build_adjacency(src, dst, edge_weight)
    A = adj.reshape(B, RP, RP)
    x3 = x.reshape(B, ROIS, H0)
    h1, h3 = _gcn_layers(A, x3, W1, b1, W3, b3)
    fc1w3 = fc1W.reshape(2 * ROIS, H1, D2)
    return _fc_head(h1, h3, fc1w3, fc1w3, fc1b, fc2W, fc2b, fc3W, fc3b,
                    bn1g, bn1b, bn2g, bn2b)


# speedup vs baseline: 96.9805x; 1.1252x over previous
import jax
import jax.numpy as jnp
from jax import lax
from jax.experimental import pallas as pl
from jax.experimental.pallas import tpu as pltpu
from jax.experimental.pallas import tpu_sc as plsc

B = 112
ROIS = 90
RP = 96
DEG = 32
EPG = ROIS * DEG
H0, H1, H3 = 128, 256, 256
D2, D3, C = 64, 16, 2
NW = 32
AW = RP * RP


def _sc_adj_body(src_hbm, dst_hbm, w_hbm, out_hbm, a_v, src_v, dst_v, w_v,
                 sem_s, sem_d, sem_w):
    wid = lax.axis_index("s") * 2 + lax.axis_index("c")
    zeros16 = jnp.zeros((16,), jnp.float32)
    for t in range(4):
        g = t * NW + wid

        @pl.when(g < B)
        def _():
            eoff = g * EPG
            cp_s = pltpu.async_copy(src_hbm.at[pl.ds(eoff, EPG)], src_v, sem_s)
            cp_d = pltpu.async_copy(dst_hbm.at[pl.ds(eoff, EPG)], dst_v, sem_d)
            cp_w = pltpu.async_copy(w_hbm.at[pl.ds(eoff, EPG)], w_v, sem_w)

            def zero_body(i, _):
                b0 = i * 256
                for k in range(16):
                    a_v[pl.ds(b0 + k * 16, 16)] = zeros16
                return 0

            lax.fori_loop(0, AW // 256, zero_body, 0)
            cp_s.wait()
            cp_d.wait()
            cp_w.wait()

            base97 = g * ROIS * (RP + 1)

            def edge_body(i, _):
                b0 = i * 64
                for k in range(4):
                    off = b0 + k * 16
                    s16 = src_v[pl.ds(off, 16)]
                    d16 = dst_v[pl.ds(off, 16)]
                    w16 = w_v[pl.ds(off, 16)]
                    idx = d16 * RP + s16 - base97
                    plsc.addupdate_scatter(a_v, [idx], w16)
                return 0

            lax.fori_loop(0, EPG // 64, edge_body, 0)

            pltpu.sync_copy(a_v, out_hbm.at[g])


def _build_adjacency(src, dst, w):
    mesh = plsc.VectorSubcoreMesh(core_axis_name="c", subcore_axis_name="s")
    kern = pl.kernel(
        _sc_adj_body,
        out_type=jax.ShapeDtypeStruct((B, AW), jnp.float32),
        mesh=mesh,
        compiler_params=pltpu.CompilerParams(needs_layout_passes=False),
        scratch_types=[
            pltpu.VMEM((AW,), jnp.float32),
            pltpu.VMEM((EPG,), jnp.int32),
            pltpu.VMEM((EPG,), jnp.int32),
            pltpu.VMEM((EPG,), jnp.float32),
            pltpu.SemaphoreType.DMA,
            pltpu.SemaphoreType.DMA,
            pltpu.SemaphoreType.DMA,
        ],
    )
    return kern(src, dst, w)


GB = 8


def _conv_body(a_ref, x_ref, w1_ref, b1_ref, w3_ref, b3_ref, h1_ref, h3_ref):
    r_i = lax.broadcasted_iota(jnp.int32, (ROIS, ROIS), 0)
    c_i = lax.broadcasted_iota(jnp.int32, (ROIS, ROIS), 1)
    eye = jnp.where(r_i == c_i, 1.0, 0.0)

    bf = jnp.bfloat16
    xg = x_ref[...].reshape(GB * ROIS, H0)
    xw1 = jnp.dot(xg.astype(bf), w1_ref[...],
                  preferred_element_type=jnp.float32)

    ans, dinvs, h1s = [], [], []
    for j in range(GB):
        A = a_ref[j, :ROIS, :ROIS]
        deg = jnp.sum(A, axis=1, keepdims=True) + 1.0
        dinv = jnp.where(deg > 0, lax.rsqrt(deg), 0.0)
        An = (A + eye).astype(bf)
        u = (dinv * xw1[j * ROIS:(j + 1) * ROIS]).astype(bf)
        h1 = jax.nn.relu(
            dinv * jnp.dot(An, u, preferred_element_type=jnp.float32)
            + b1_ref[...]).astype(bf)
        h1_ref[:, j, :] = h1
        ans.append(An)
        dinvs.append(dinv)
        h1s.append(h1)

    xw2 = jnp.dot(jnp.concatenate(h1s, axis=0), w3_ref[...],
                  preferred_element_type=jnp.float32)
    for j in range(GB):
        u = (dinvs[j] * xw2[j * ROIS:(j + 1) * ROIS]).astype(bf)
        h3 = jax.nn.relu(
            dinvs[j] * jnp.dot(ans[j], u, preferred_element_type=jnp.float32)
            + b3_ref[...]).astype(bf)
        h3_ref[:, j, :] = h3


def _gcn_layers(A, x3, W1, b1, W3, b3):
    h_shape = jax.ShapeDtypeStruct((ROIS, B, H1), jnp.bfloat16)
    return pl.pallas_call(
        _conv_body,
        grid=(B // GB,),
        in_specs=[
            pl.BlockSpec((GB, RP, RP), lambda g: (g, 0, 0)),
            pl.BlockSpec((GB, ROIS, H0), lambda g: (g, 0, 0)),
            pl.BlockSpec((H0, H1), lambda g: (0, 0)),
            pl.BlockSpec((1, H1), lambda g: (0, 0)),
            pl.BlockSpec((H1, H3), lambda g: (0, 0)),
            pl.BlockSpec((1, H3), lambda g: (0, 0)),
        ],
        out_specs=[
            pl.BlockSpec((ROIS, GB, H1), lambda g: (0, g, 0)),
            pl.BlockSpec((ROIS, GB, H3), lambda g: (0, g, 0)),
        ],
        out_shape=[h_shape, h_shape],
    )(A, x3, W1.astype(jnp.bfloat16), b1.reshape(1, -1),
      W3.astype(jnp.bfloat16), b3.reshape(1, -1))


RC = 6


def _head_body(h1_ref, h3_ref, wt_ref, wb_ref, fc1b_ref, fc2w_ref, fc2b_ref,
               fc3w_ref, fc3b_ref, g1_ref, bb1_ref, g2_ref, bb2_ref,
               out_ref, acc):
    r = pl.program_id(0)

    @pl.when(r == 0)
    def _():
        acc[...] = jnp.zeros_like(acc)

    bf = jnp.bfloat16
    wtb = wt_ref[...].astype(bf)
    wbb = wb_ref[...].astype(bf)
    part = jnp.zeros((B, D2), jnp.float32)
    for j in range(RC):
        h1r = h1_ref[j]
        h3r = h3_ref[j]
        part += (jnp.dot(h1r, wtb[j], preferred_element_type=jnp.float32)
                 + jnp.dot(h3r, wbb[j], preferred_element_type=jnp.float32))
    acc[...] += part

    @pl.when(r == ROIS // RC - 1)
    def _():
        y = jax.nn.relu(acc[...] + fc1b_ref[...])
        mu = jnp.mean(y, axis=0, keepdims=True)
        var = jnp.mean((y - mu) ** 2, axis=0, keepdims=True)
        y = g1_ref[...] * (y - mu) / jnp.sqrt(var + 1e-5) + bb1_ref[...]
        y = jax.nn.relu(
            jnp.dot(y, fc2w_ref[...], preferred_element_type=jnp.float32)
            + fc2b_ref[...])
        mu2 = jnp.mean(y, axis=0, keepdims=True)
        var2 = jnp.mean((y - mu2) ** 2, axis=0, keepdims=True)
        y = g2_ref[...] * (y - mu2) / jnp.sqrt(var2 + 1e-5) + bb2_ref[...]
        z = (jnp.dot(y, fc3w_ref[...], preferred_element_type=jnp.float32)
             + fc3b_ref[...])
        m = jnp.max(z, axis=1, keepdims=True)
        lse = m + jnp.log(jnp.sum(jnp.exp(z - m), axis=1, keepdims=True))
        out_ref[...] = z - lse


def _fc_head(h1, h3, wt, wb, fc1b, fc2W, fc2b, fc3W, fc3b,
             bn1g, bn1b, bn2g, bn2b):
    return pl.pallas_call(
        _head_body,
        grid=(ROIS // RC,),
        in_specs=[
            pl.BlockSpec((RC, B, H1), lambda r: (r, 0, 0)),
            pl.BlockSpec((RC, B, H3), lambda r: (r, 0, 0)),
            pl.BlockSpec((RC, H1, D2), lambda r: (r, 0, 0)),
            pl.BlockSpec((RC, H3, D2), lambda r: (r + ROIS // RC, 0, 0)),
            pl.BlockSpec((1, D2), lambda r: (0, 0)),
            pl.BlockSpec((D2, D3), lambda r: (0, 0)),
            pl.BlockSpec((1, D3), lambda r: (0, 0)),
            pl.BlockSpec((D3, C), lambda r: (0, 0)),
            pl.BlockSpec((1, C), lambda r: (0, 0)),
            pl.BlockSpec((1, D2), lambda r: (0, 0)),
            pl.BlockSpec((1, D2), lambda r: (0, 0)),
            pl.BlockSpec((1, D3), lambda r: (0, 0)),
            pl.BlockSpec((1, D3), lambda r: (0, 0)),
        ],
        out_specs=pl.BlockSpec((B, C), lambda r: (0, 0)),
        out_shape=jax.ShapeDtypeStruct((B, C), jnp.float32),
        scratch_shapes=[pltpu.VMEM((B, D2), jnp.float32)],
    )(h1, h3, wt, wb, fc1b.reshape(1, -1), fc2W, fc2b.reshape(1, -1),
      fc3W, fc3b.reshape(1, -1), bn1g.reshape(1, -1), bn1b.reshape(1, -1),
      bn2g.reshape(1, -1), bn2b.reshape(1, -1))


def kernel(x, edge_index, edge_weight, batch, W1, b1, W3, b3,
           fc1W, fc1b, fc2W, fc2b, fc3W, fc3b, bn1g, bn1b, bn2g, bn2b):
    src = edge_index[0]
    dst = edge_index[1]
    adj = _build_adjacency(src, dst, edge_weight)
    A = adj.reshape(B, RP, RP)
    x3 = x.reshape(B, ROIS, H0)
    h1, h3 = _gcn_layers(A, x3, W1, b1, W3, b3)
    fc1w3 = fc1W.reshape(2 * ROIS, H1, D2)
    return _fc_head(h1, h3, fc1w3, fc1w3, fc1b, fc2W, fc2b, fc3W, fc3b,
                    bn1g, bn1b, bn2g, bn2b)


# speedup vs baseline: 97.4197x; 1.0045x over previous
import jax
import jax.numpy as jnp
from jax import lax
from jax.experimental import pallas as pl
from jax.experimental.pallas import tpu as pltpu
from jax.experimental.pallas import tpu_sc as plsc

B = 112
ROIS = 90
RP = 96
DEG = 32
EPG = ROIS * DEG
H0, H1, H3 = 128, 256, 256
D2, D3, C = 64, 16, 2
NW = 32
AW = RP * RP


def _sc_adj_body(src_hbm, dst_hbm, w_hbm, out_hbm, a_v, src_v, dst_v, w_v,
                 sem_s, sem_d, sem_w):
    wid = lax.axis_index("s") * 2 + lax.axis_index("c")
    zeros16 = jnp.zeros((16,), jnp.float32)
    for t in range(4):
        g = t * NW + wid

        @pl.when(g < B)
        def _():
            eoff = g * EPG
            cp_s = pltpu.async_copy(src_hbm.at[pl.ds(eoff, EPG)], src_v, sem_s)
            cp_d = pltpu.async_copy(dst_hbm.at[pl.ds(eoff, EPG)], dst_v, sem_d)
            cp_w = pltpu.async_copy(w_hbm.at[pl.ds(eoff, EPG)], w_v, sem_w)

            def zero_body(i, _):
                b0 = i * 256
                for k in range(16):
                    a_v[pl.ds(b0 + k * 16, 16)] = zeros16
                return 0

            lax.fori_loop(0, AW // 256, zero_body, 0)
            cp_s.wait()
            cp_d.wait()
            cp_w.wait()

            base97 = g * ROIS * (RP + 1)

            def edge_body(i, _):
                b0 = i * 64
                for k in range(4):
                    off = b0 + k * 16
                    s16 = src_v[pl.ds(off, 16)]
                    d16 = dst_v[pl.ds(off, 16)]
                    w16 = w_v[pl.ds(off, 16)]
                    idx = d16 * RP + s16 - base97
                    plsc.addupdate_scatter(a_v, [idx], w16)
                return 0

            lax.fori_loop(0, EPG // 64, edge_body, 0)

            pltpu.sync_copy(a_v, out_hbm.at[g])


def _build_adjacency(src, dst, w):
    mesh = plsc.VectorSubcoreMesh(core_axis_name="c", subcore_axis_name="s")
    kern = pl.kernel(
        _sc_adj_body,
        out_type=jax.ShapeDtypeStruct((B, AW), jnp.float32),
        mesh=mesh,
        compiler_params=pltpu.CompilerParams(needs_layout_passes=False),
        scratch_types=[
            pltpu.VMEM((AW,), jnp.float32),
            pltpu.VMEM((EPG,), jnp.int32),
            pltpu.VMEM((EPG,), jnp.int32),
            pltpu.VMEM((EPG,), jnp.float32),
            pltpu.SemaphoreType.DMA,
            pltpu.SemaphoreType.DMA,
            pltpu.SemaphoreType.DMA,
        ],
    )
    return kern(src, dst, w)


GB = 8


def _xw1_body(x_ref, w1_ref, xw1_ref):
    xg = x_ref[...].astype(jnp.bfloat16)
    xw1_ref[...] = jnp.dot(
        xg, w1_ref[...],
        preferred_element_type=jnp.float32).astype(jnp.bfloat16)


def _xw1(x2, W1):
    n = B * ROIS
    return pl.pallas_call(
        _xw1_body,
        grid=(4,),
        in_specs=[
            pl.BlockSpec((n // 4, H0), lambda i: (i, 0)),
            pl.BlockSpec((H0, H1), lambda i: (0, 0)),
        ],
        out_specs=pl.BlockSpec((n // 4, H1), lambda i: (i, 0)),
        out_shape=jax.ShapeDtypeStruct((n, H1), jnp.bfloat16),
    )(x2, W1.astype(jnp.bfloat16))


def _conv_body(a_ref, xw1_ref, b1_ref, w3_ref, b3_ref, h1_ref, h3_ref):
    r_i = lax.broadcasted_iota(jnp.int32, (ROIS, ROIS), 0)
    c_i = lax.broadcasted_iota(jnp.int32, (ROIS, ROIS), 1)
    eye = jnp.where(r_i == c_i, 1.0, 0.0)

    bf = jnp.bfloat16
    xw1 = xw1_ref[...]

    ans, dinvs, h1s = [], [], []
    for j in range(GB):
        A = a_ref[j, :ROIS, :ROIS]
        deg = jnp.sum(A, axis=1, keepdims=True) + 1.0
        dinv = jnp.where(deg > 0, lax.rsqrt(deg), 0.0)
        An = (A + eye).astype(bf)
        u = (dinv * xw1[j * ROIS:(j + 1) * ROIS]).astype(bf)
        h1 = jax.nn.relu(
            dinv * jnp.dot(An, u, preferred_element_type=jnp.float32)
            + b1_ref[...]).astype(bf)
        h1_ref[:, j, :] = h1
        ans.append(An)
        dinvs.append(dinv)
        h1s.append(h1)

    xw2 = jnp.dot(jnp.concatenate(h1s, axis=0), w3_ref[...],
                  preferred_element_type=jnp.float32)
    for j in range(GB):
        u = (dinvs[j] * xw2[j * ROIS:(j + 1) * ROIS]).astype(bf)
        h3 = jax.nn.relu(
            dinvs[j] * jnp.dot(ans[j], u, preferred_element_type=jnp.float32)
            + b3_ref[...]).astype(bf)
        h3_ref[:, j, :] = h3


def _gcn_layers(A, xw1, b1, W3, b3):
    h_shape = jax.ShapeDtypeStruct((ROIS, B, H1), jnp.bfloat16)
    return pl.pallas_call(
        _conv_body,
        grid=(B // GB,),
        in_specs=[
            pl.BlockSpec((GB, RP, RP), lambda g: (g, 0, 0)),
            pl.BlockSpec((GB * ROIS, H1), lambda g: (g, 0)),
            pl.BlockSpec((1, H1), lambda g: (0, 0)),
            pl.BlockSpec((H1, H3), lambda g: (0, 0)),
            pl.BlockSpec((1, H3), lambda g: (0, 0)),
        ],
        out_specs=[
            pl.BlockSpec((ROIS, GB, H1), lambda g: (0, g, 0)),
            pl.BlockSpec((ROIS, GB, H3), lambda g: (0, g, 0)),
        ],
        out_shape=[h_shape, h_shape],
    )(A, xw1, b1.reshape(1, -1), W3.astype(jnp.bfloat16), b3.reshape(1, -1))


RC = 6


def _head_body(h1_ref, h3_ref, wt_ref, wb_ref, fc1b_ref, fc2w_ref, fc2b_ref,
               fc3w_ref, fc3b_ref, g1_ref, bb1_ref, g2_ref, bb2_ref,
               out_ref, acc):
    r = pl.program_id(0)

    @pl.when(r == 0)
    def _():
        acc[...] = jnp.zeros_like(acc)

    bf = jnp.bfloat16
    wtb = wt_ref[...].astype(bf)
    wbb = wb_ref[...].astype(bf)
    part = jnp.zeros((B, D2), jnp.float32)
    for j in range(RC):
        h1r = h1_ref[j]
        h3r = h3_ref[j]
        part += (jnp.dot(h1r, wtb[j], preferred_element_type=jnp.float32)
                 + jnp.dot(h3r, wbb[j], preferred_element_type=jnp.float32))
    acc[...] += part

    @pl.when(r == ROIS // RC - 1)
    def _():
        y = jax.nn.relu(acc[...] + fc1b_ref[...])
        mu = jnp.mean(y, axis=0, keepdims=True)
        var = jnp.mean((y - mu) ** 2, axis=0, keepdims=True)
        y = g1_ref[...] * (y - mu) / jnp.sqrt(var + 1e-5) + bb1_ref[...]
        y = jax.nn.relu(
            jnp.dot(y, fc2w_ref[...], preferred_element_type=jnp.float32)
            + fc2b_ref[...])
        mu2 = jnp.mean(y, axis=0, keepdims=True)
        var2 = jnp.mean((y - mu2) ** 2, axis=0, keepdims=True)
        y = g2_ref[...] * (y - mu2) / jnp.sqrt(var2 + 1e-5) + bb2_ref[...]
        z = (jnp.dot(y, fc3w_ref[...], preferred_element_type=jnp.float32)
             + fc3b_ref[...])
        m = jnp.max(z, axis=1, keepdims=True)
        lse = m + jnp.log(jnp.sum(jnp.exp(z - m), axis=1, keepdims=True))
        out_ref[...] = z - lse


def _fc_head(h1, h3, wt, wb, fc1b, fc2W, fc2b, fc3W, fc3b,
             bn1g, bn1b, bn2g, bn2b):
    return pl.pallas_call(
        _head_body,
        grid=(ROIS // RC,),
        in_specs=[
            pl.BlockSpec((RC, B, H1), lambda r: (r, 0, 0)),
            pl.BlockSpec((RC, B, H3), lambda r: (r, 0, 0)),
            pl.BlockSpec((RC, H1, D2), lambda r: (r, 0, 0)),
            pl.BlockSpec((RC, H3, D2), lambda r: (r + ROIS // RC, 0, 0)),
            pl.BlockSpec((1, D2), lambda r: (0, 0)),
            pl.BlockSpec((D2, D3), lambda r: (0, 0)),
            pl.BlockSpec((1, D3), lambda r: (0, 0)),
            pl.BlockSpec((D3, C), lambda r: (0, 0)),
            pl.BlockSpec((1, C), lambda r: (0, 0)),
            pl.BlockSpec((1, D2), lambda r: (0, 0)),
            pl.BlockSpec((1, D2), lambda r: (0, 0)),
            pl.BlockSpec((1, D3), lambda r: (0, 0)),
            pl.BlockSpec((1, D3), lambda r: (0, 0)),
        ],
        out_specs=pl.BlockSpec((B, C), lambda r: (0, 0)),
        out_shape=jax.ShapeDtypeStruct((B, C), jnp.float32),
        scratch_shapes=[pltpu.VMEM((B, D2), jnp.float32)],
    )(h1, h3, wt, wb, fc1b.reshape(1, -1), fc2W, fc2b.reshape(1, -1),
      fc3W, fc3b.reshape(1, -1), bn1g.reshape(1, -1), bn1b.reshape(1, -1),
      bn2g.reshape(1, -1), bn2b.reshape(1, -1))


def kernel(x, edge_index, edge_weight, batch, W1, b1, W3, b3,
           fc1W, fc1b, fc2W, fc2b, fc3W, fc3b, bn1g, bn1b, bn2g, bn2b):
    src = edge_index[0]
    dst = edge_index[1]
    xw1 = _xw1(x, W1)
    adj = _build_adjacency(src, dst, edge_weight)
    A = adj.reshape(B, RP, RP)
    h1, h3 = _gcn_layers(A, xw1, b1, W3, b3)
    fc1w3 = fc1W.reshape(2 * ROIS, H1, D2)
    return _fc_head(h1, h3, fc1w3, fc1w3, fc1b, fc2W, fc2b, fc3W, fc3b,
                    bn1g, bn1b, bn2g, bn2b)


# speedup vs baseline: 101.0268x; 1.0370x over previous
import jax
import jax.numpy as jnp
from jax import lax
from jax.experimental import pallas as pl
from jax.experimental.pallas import tpu as pltpu
from jax.experimental.pallas import tpu_sc as plsc

B = 112
ROIS = 90
RP = 96
DEG = 32
EPG = ROIS * DEG
H0, H1, H3 = 128, 256, 256
D2, D3, C = 64, 16, 2
NW = 32
AW = RP * RP


def _sc_adj_body(src_hbm, dst_hbm, w_hbm, out_hbm, a_v, src_v, dst_v, w_v,
                 sem_s, sem_d, sem_w):
    wid = lax.axis_index("s") * 2 + lax.axis_index("c")
    zeros16 = jnp.zeros((16,), jnp.float32)
    for t in range(4):
        g = t * NW + wid

        @pl.when(g < B)
        def _():
            eoff = g * EPG
            cp_s = pltpu.async_copy(src_hbm.at[pl.ds(eoff, EPG)], src_v, sem_s)
            cp_d = pltpu.async_copy(dst_hbm.at[pl.ds(eoff, EPG)], dst_v, sem_d)
            cp_w = pltpu.async_copy(w_hbm.at[pl.ds(eoff, EPG)], w_v, sem_w)

            def zero_body(i, _):
                b0 = i * 256
                for k in range(16):
                    a_v[pl.ds(b0 + k * 16, 16)] = zeros16
                return 0

            lax.fori_loop(0, AW // 256, zero_body, 0)
            cp_s.wait()
            cp_d.wait()
            cp_w.wait()

            base97 = g * ROIS * (RP + 1)

            def edge_body(i, _):
                b0 = i * 64
                for k in range(4):
                    off = b0 + k * 16
                    s16 = src_v[pl.ds(off, 16)]
                    d16 = dst_v[pl.ds(off, 16)]
                    w16 = w_v[pl.ds(off, 16)]
                    idx = d16 * RP + s16 - base97
                    plsc.addupdate_scatter(a_v, [idx], w16)
                return 0

            lax.fori_loop(0, EPG // 64, edge_body, 0)

            pltpu.sync_copy(a_v, out_hbm.at[g])


def _build_adjacency(src, dst, w):
    mesh = plsc.VectorSubcoreMesh(core_axis_name="c", subcore_axis_name="s")
    kern = pl.kernel(
        _sc_adj_body,
        out_type=jax.ShapeDtypeStruct((B, AW), jnp.float32),
        mesh=mesh,
        compiler_params=pltpu.CompilerParams(needs_layout_passes=False),
        scratch_types=[
            pltpu.VMEM((AW,), jnp.float32),
            pltpu.VMEM((EPG,), jnp.int32),
            pltpu.VMEM((EPG,), jnp.int32),
            pltpu.VMEM((EPG,), jnp.float32),
            pltpu.SemaphoreType.DMA,
            pltpu.SemaphoreType.DMA,
            pltpu.SemaphoreType.DMA,
        ],
    )
    return kern(src, dst, w)


GB = 16


def _xw1_body(x_ref, w1_ref, xw1_ref):
    xg = x_ref[...].astype(jnp.bfloat16)
    xw1_ref[...] = jnp.dot(
        xg, w1_ref[...],
        preferred_element_type=jnp.float32).astype(jnp.bfloat16)


def _xw1(x2, W1):
    n = B * ROIS
    return pl.pallas_call(
        _xw1_body,
        grid=(4,),
        in_specs=[
            pl.BlockSpec((n // 4, H0), lambda i: (i, 0)),
            pl.BlockSpec((H0, H1), lambda i: (0, 0)),
        ],
        out_specs=pl.BlockSpec((n // 4, H1), lambda i: (i, 0)),
        out_shape=jax.ShapeDtypeStruct((n, H1), jnp.bfloat16),
    )(x2, W1.astype(jnp.bfloat16))


def _conv_body(a_ref, xw1_ref, b1_ref, w3_ref, b3_ref, h1_ref, h3_ref):
    r_i = lax.broadcasted_iota(jnp.int32, (ROIS, ROIS), 0)
    c_i = lax.broadcasted_iota(jnp.int32, (ROIS, ROIS), 1)
    eye = jnp.where(r_i == c_i, 1.0, 0.0)

    bf = jnp.bfloat16
    xw1 = xw1_ref[...]

    ans, dinvs, h1s = [], [], []
    for j in range(GB):
        A = a_ref[j, :ROIS, :ROIS]
        deg = jnp.sum(A, axis=1, keepdims=True) + 1.0
        dinv = jnp.where(deg > 0, lax.rsqrt(deg), 0.0)
        An = (A + eye).astype(bf)
        u = (dinv * xw1[j * ROIS:(j + 1) * ROIS]).astype(bf)
        h1 = jax.nn.relu(
            dinv * jnp.dot(An, u, preferred_element_type=jnp.float32)
            + b1_ref[...]).astype(bf)
        h1_ref[:, j, :] = h1
        ans.append(An)
        dinvs.append(dinv)
        h1s.append(h1)

    xw2 = jnp.dot(jnp.concatenate(h1s, axis=0), w3_ref[...],
                  preferred_element_type=jnp.float32)
    for j in range(GB):
        u = (dinvs[j] * xw2[j * ROIS:(j + 1) * ROIS]).astype(bf)
        h3 = jax.nn.relu(
            dinvs[j] * jnp.dot(ans[j], u, preferred_element_type=jnp.float32)
            + b3_ref[...]).astype(bf)
        h3_ref[:, j, :] = h3


def _gcn_layers(A, xw1, b1, W3, b3):
    h_shape = jax.ShapeDtypeStruct((ROIS, B, H1), jnp.bfloat16)
    return pl.pallas_call(
        _conv_body,
        grid=(B // GB,),
        in_specs=[
            pl.BlockSpec((GB, RP, RP), lambda g: (g, 0, 0)),
            pl.BlockSpec((GB * ROIS, H1), lambda g: (g, 0)),
            pl.BlockSpec((1, H1), lambda g: (0, 0)),
            pl.BlockSpec((H1, H3), lambda g: (0, 0)),
            pl.BlockSpec((1, H3), lambda g: (0, 0)),
        ],
        out_specs=[
            pl.BlockSpec((ROIS, GB, H1), lambda g: (0, g, 0)),
            pl.BlockSpec((ROIS, GB, H3), lambda g: (0, g, 0)),
        ],
        out_shape=[h_shape, h_shape],
    )(A, xw1, b1.reshape(1, -1), W3.astype(jnp.bfloat16), b3.reshape(1, -1))


RC = 10


def _head_body(h1_ref, h3_ref, wt_ref, wb_ref, fc1b_ref, fc2w_ref, fc2b_ref,
               fc3w_ref, fc3b_ref, g1_ref, bb1_ref, g2_ref, bb2_ref,
               out_ref, acc):
    r = pl.program_id(0)

    @pl.when(r == 0)
    def _():
        acc[...] = jnp.zeros_like(acc)

    bf = jnp.bfloat16
    wtb = wt_ref[...].astype(bf)
    wbb = wb_ref[...].astype(bf)
    part = jnp.zeros((B, D2), jnp.float32)
    for j in range(RC):
        h1r = h1_ref[j]
        h3r = h3_ref[j]
        part += (jnp.dot(h1r, wtb[j], preferred_element_type=jnp.float32)
                 + jnp.dot(h3r, wbb[j], preferred_element_type=jnp.float32))
    acc[...] += part

    @pl.when(r == ROIS // RC - 1)
    def _():
        y = jax.nn.relu(acc[...] + fc1b_ref[...])
        mu = jnp.mean(y, axis=0, keepdims=True)
        var = jnp.mean((y - mu) ** 2, axis=0, keepdims=True)
        y = g1_ref[...] * (y - mu) / jnp.sqrt(var + 1e-5) + bb1_ref[...]
        y = jax.nn.relu(
            jnp.dot(y, fc2w_ref[...], preferred_element_type=jnp.float32)
            + fc2b_ref[...])
        mu2 = jnp.mean(y, axis=0, keepdims=True)
        var2 = jnp.mean((y - mu2) ** 2, axis=0, keepdims=True)
        y = g2_ref[...] * (y - mu2) / jnp.sqrt(var2 + 1e-5) + bb2_ref[...]
        z = (jnp.dot(y, fc3w_ref[...], preferred_element_type=jnp.float32)
             + fc3b_ref[...])
        m = jnp.max(z, axis=1, keepdims=True)
        lse = m + jnp.log(jnp.sum(jnp.exp(z - m), axis=1, keepdims=True))
        out_ref[...] = z - lse


def _fc_head(h1, h3, wt, wb, fc1b, fc2W, fc2b, fc3W, fc3b,
             bn1g, bn1b, bn2g, bn2b):
    return pl.pallas_call(
        _head_body,
        grid=(ROIS // RC,),
        in_specs=[
            pl.BlockSpec((RC, B, H1), lambda r: (r, 0, 0)),
            pl.BlockSpec((RC, B, H3), lambda r: (r, 0, 0)),
            pl.BlockSpec((RC, H1, D2), lambda r: (r, 0, 0)),
            pl.BlockSpec((RC, H3, D2), lambda r: (r + ROIS // RC, 0, 0)),
            pl.BlockSpec((1, D2), lambda r: (0, 0)),
            pl.BlockSpec((D2, D3), lambda r: (0, 0)),
            pl.BlockSpec((1, D3), lambda r: (0, 0)),
            pl.BlockSpec((D3, C), lambda r: (0, 0)),
            pl.BlockSpec((1, C), lambda r: (0, 0)),
            pl.BlockSpec((1, D2), lambda r: (0, 0)),
            pl.BlockSpec((1, D2), lambda r: (0, 0)),
            pl.BlockSpec((1, D3), lambda r: (0, 0)),
            pl.BlockSpec((1, D3), lambda r: (0, 0)),
        ],
        out_specs=pl.BlockSpec((B, C), lambda r: (0, 0)),
        out_shape=jax.ShapeDtypeStruct((B, C), jnp.float32),
        scratch_shapes=[pltpu.VMEM((B, D2), jnp.float32)],
    )(h1, h3, wt, wb, fc1b.reshape(1, -1), fc2W, fc2b.reshape(1, -1),
      fc3W, fc3b.reshape(1, -1), bn1g.reshape(1, -1), bn1b.reshape(1, -1),
      bn2g.reshape(1, -1), bn2b.reshape(1, -1))


def kernel(x, edge_index, edge_weight, batch, W1, b1, W3, b3,
           fc1W, fc1b, fc2W, fc2b, fc3W, fc3b, bn1g, bn1b, bn2g, bn2b):
    src = edge_index[0]
    dst = edge_index[1]
    xw1 = _xw1(x, W1)
    adj = _build_adjacency(src, dst, edge_weight)
    A = adj.reshape(B, RP, RP)
    h1, h3 = _gcn_layers(A, xw1, b1, W3, b3)
    fc1w3 = fc1W.reshape(2 * ROIS, H1, D2)
    return _fc_head(h1, h3, fc1w3, fc1w3, fc1b, fc2W, fc2b, fc3W, fc3b,
                    bn1g, bn1b, bn2g, bn2b)


# speedup vs baseline: 103.2047x; 1.0216x over previous
import jax
import jax.numpy as jnp
from jax import lax
from jax.experimental import pallas as pl
from jax.experimental.pallas import tpu as pltpu
from jax.experimental.pallas import tpu_sc as plsc

B = 112
ROIS = 90
RP = 96
DEG = 32
EPG = ROIS * DEG
H0, H1, H3 = 128, 256, 256
D2, D3, C = 64, 16, 2
NW = 32
AW = RP * RP


def _sc_adj_body(src_hbm, dst_hbm, w_hbm, out_hbm, a_v, src_v, dst_v, w_v,
                 sem_s, sem_d, sem_w):
    wid = lax.axis_index("s") * 2 + lax.axis_index("c")
    zeros16 = jnp.zeros((16,), jnp.float32)
    for t in range(4):
        g = t * NW + wid

        @pl.when(g < B)
        def _():
            eoff = g * EPG
            cp_s = pltpu.async_copy(src_hbm.at[pl.ds(eoff, EPG)], src_v, sem_s)
            cp_d = pltpu.async_copy(dst_hbm.at[pl.ds(eoff, EPG)], dst_v, sem_d)
            cp_w = pltpu.async_copy(w_hbm.at[pl.ds(eoff, EPG)], w_v, sem_w)

            def zero_body(i, _):
                b0 = i * 256
                for k in range(16):
                    a_v[pl.ds(b0 + k * 16, 16)] = zeros16
                return 0

            lax.fori_loop(0, AW // 256, zero_body, 0)
            cp_s.wait()
            cp_d.wait()
            cp_w.wait()

            base97 = g * ROIS * (RP + 1)

            def edge_body(i, _):
                b0 = i * 64
                for k in range(4):
                    off = b0 + k * 16
                    s16 = src_v[pl.ds(off, 16)]
                    d16 = dst_v[pl.ds(off, 16)]
                    w16 = w_v[pl.ds(off, 16)]
                    idx = d16 * RP + s16 - base97
                    plsc.addupdate_scatter(a_v, [idx], w16)
                return 0

            lax.fori_loop(0, EPG // 64, edge_body, 0)

            pltpu.sync_copy(a_v, out_hbm.at[g])


def _build_adjacency(src, dst, w):
    mesh = plsc.VectorSubcoreMesh(core_axis_name="c", subcore_axis_name="s")
    kern = pl.kernel(
        _sc_adj_body,
        out_type=jax.ShapeDtypeStruct((B, AW), jnp.float32),
        mesh=mesh,
        compiler_params=pltpu.CompilerParams(needs_layout_passes=False),
        scratch_types=[
            pltpu.VMEM((AW,), jnp.float32),
            pltpu.VMEM((EPG,), jnp.int32),
            pltpu.VMEM((EPG,), jnp.int32),
            pltpu.VMEM((EPG,), jnp.float32),
            pltpu.SemaphoreType.DMA,
            pltpu.SemaphoreType.DMA,
            pltpu.SemaphoreType.DMA,
        ],
    )
    return kern(src, dst, w)


GB = 16


def _xw1_body(x_ref, w1_ref, xw1_ref):
    xg = x_ref[...].astype(jnp.bfloat16)
    xw1_ref[...] = jnp.dot(
        xg, w1_ref[...],
        preferred_element_type=jnp.float32).astype(jnp.bfloat16)


def _xw1(x2, W1):
    n = B * ROIS
    return pl.pallas_call(
        _xw1_body,
        grid=(4,),
        in_specs=[
            pl.BlockSpec((n // 4, H0), lambda i: (i, 0)),
            pl.BlockSpec((H0, H1), lambda i: (0, 0)),
        ],
        out_specs=pl.BlockSpec((n // 4, H1), lambda i: (i, 0)),
        out_shape=jax.ShapeDtypeStruct((n, H1), jnp.bfloat16),
    )(x2, W1.astype(jnp.bfloat16))


def _conv_body(a_ref, xw1_ref, b1_ref, w3_ref, b3_ref, h1_ref, h3_ref):
    r_i = lax.broadcasted_iota(jnp.int32, (ROIS, ROIS), 0)
    c_i = lax.broadcasted_iota(jnp.int32, (ROIS, ROIS), 1)
    eye = jnp.where(r_i == c_i, 1.0, 0.0)

    bf = jnp.bfloat16
    xw1 = xw1_ref[...]

    ans, dinvs, h1s = [], [], []
    for j in range(GB):
        A = a_ref[j, :ROIS, :ROIS]
        deg = jnp.sum(A, axis=1, keepdims=True) + 1.0
        dinv = jnp.where(deg > 0, lax.rsqrt(deg), 0.0)
        An = (A + eye).astype(bf)
        u = (dinv * xw1[j * ROIS:(j + 1) * ROIS]).astype(bf)
        h1 = jax.nn.relu(
            dinv * jnp.dot(An, u, preferred_element_type=jnp.float32)
            + b1_ref[...]).astype(bf)
        h1_ref[:, j, :] = h1
        ans.append(An)
        dinvs.append(dinv)
        h1s.append(h1)

    xw2 = jnp.dot(jnp.concatenate(h1s, axis=0), w3_ref[...],
                  preferred_element_type=jnp.float32)
    for j in range(GB):
        u = (dinvs[j] * xw2[j * ROIS:(j + 1) * ROIS]).astype(bf)
        h3 = jax.nn.relu(
            dinvs[j] * jnp.dot(ans[j], u, preferred_element_type=jnp.float32)
            + b3_ref[...]).astype(bf)
        h3_ref[:, j, :] = h3


def _gcn_layers(A, xw1, b1, W3, b3):
    h_shape = jax.ShapeDtypeStruct((ROIS, B, H1), jnp.bfloat16)
    return pl.pallas_call(
        _conv_body,
        grid=(B // GB,),
        in_specs=[
            pl.BlockSpec((GB, RP, RP), lambda g: (g, 0, 0)),
            pl.BlockSpec((GB * ROIS, H1), lambda g: (g, 0)),
            pl.BlockSpec((1, H1), lambda g: (0, 0)),
            pl.BlockSpec((H1, H3), lambda g: (0, 0)),
            pl.BlockSpec((1, H3), lambda g: (0, 0)),
        ],
        out_specs=[
            pl.BlockSpec((ROIS, GB, H1), lambda g: (0, g, 0)),
            pl.BlockSpec((ROIS, GB, H3), lambda g: (0, g, 0)),
        ],
        out_shape=[h_shape, h_shape],
    )(A, xw1, b1.reshape(1, -1), W3.astype(jnp.bfloat16), b3.reshape(1, -1))


RC = 18


def _head_body(h1_ref, h3_ref, wt_ref, wb_ref, fc1b_ref, fc2w_ref, fc2b_ref,
               fc3w_ref, fc3b_ref, g1_ref, bb1_ref, g2_ref, bb2_ref,
               out_ref, acc):
    r = pl.program_id(0)

    @pl.when(r == 0)
    def _():
        acc[...] = jnp.zeros_like(acc)

    bf = jnp.bfloat16
    wtb = wt_ref[...].astype(bf)
    wbb = wb_ref[...].astype(bf)
    part = jnp.zeros((B, D2), jnp.float32)
    for j in range(RC):
        h1r = h1_ref[j]
        h3r = h3_ref[j]
        part += (jnp.dot(h1r, wtb[j], preferred_element_type=jnp.float32)
                 + jnp.dot(h3r, wbb[j], preferred_element_type=jnp.float32))
    acc[...] += part

    @pl.when(r == ROIS // RC - 1)
    def _():
        y = jax.nn.relu(acc[...] + fc1b_ref[...])
        mu = jnp.mean(y, axis=0, keepdims=True)
        var = jnp.mean((y - mu) ** 2, axis=0, keepdims=True)
        y = g1_ref[...] * (y - mu) / jnp.sqrt(var + 1e-5) + bb1_ref[...]
        y = jax.nn.relu(
            jnp.dot(y, fc2w_ref[...], preferred_element_type=jnp.float32)
            + fc2b_ref[...])
        mu2 = jnp.mean(y, axis=0, keepdims=True)
        var2 = jnp.mean((y - mu2) ** 2, axis=0, keepdims=True)
        y = g2_ref[...] * (y - mu2) / jnp.sqrt(var2 + 1e-5) + bb2_ref[...]
        z = (jnp.dot(y, fc3w_ref[...], preferred_element_type=jnp.float32)
             + fc3b_ref[...])
        m = jnp.max(z, axis=1, keepdims=True)
        lse = m + jnp.log(jnp.sum(jnp.exp(z - m), axis=1, keepdims=True))
        out_ref[...] = z - lse


def _fc_head(h1, h3, wt, wb, fc1b, fc2W, fc2b, fc3W, fc3b,
             bn1g, bn1b, bn2g, bn2b):
    return pl.pallas_call(
        _head_body,
        grid=(ROIS // RC,),
        in_specs=[
            pl.BlockSpec((RC, B, H1), lambda r: (r, 0, 0)),
            pl.BlockSpec((RC, B, H3), lambda r: (r, 0, 0)),
            pl.BlockSpec((RC, H1, D2), lambda r: (r, 0, 0)),
            pl.BlockSpec((RC, H3, D2), lambda r: (r + ROIS // RC, 0, 0)),
            pl.BlockSpec((1, D2), lambda r: (0, 0)),
            pl.BlockSpec((D2, D3), lambda r: (0, 0)),
            pl.BlockSpec((1, D3), lambda r: (0, 0)),
            pl.BlockSpec((D3, C), lambda r: (0, 0)),
            pl.BlockSpec((1, C), lambda r: (0, 0)),
            pl.BlockSpec((1, D2), lambda r: (0, 0)),
            pl.BlockSpec((1, D2), lambda r: (0, 0)),
            pl.BlockSpec((1, D3), lambda r: (0, 0)),
            pl.BlockSpec((1, D3), lambda r: (0, 0)),
        ],
        out_specs=pl.BlockSpec((B, C), lambda r: (0, 0)),
        out_shape=jax.ShapeDtypeStruct((B, C), jnp.float32),
        scratch_shapes=[pltpu.VMEM((B, D2), jnp.float32)],
    )(h1, h3, wt, wb, fc1b.reshape(1, -1), fc2W, fc2b.reshape(1, -1),
      fc3W, fc3b.reshape(1, -1), bn1g.reshape(1, -1), bn1b.reshape(1, -1),
      bn2g.reshape(1, -1), bn2b.reshape(1, -1))


def kernel(x, edge_index, edge_weight, batch, W1, b1, W3, b3,
           fc1W, fc1b, fc2W, fc2b, fc3W, fc3b, bn1g, bn1b, bn2g, bn2b):
    src = edge_index[0]
    dst = edge_index[1]
    xw1 = _xw1(x, W1)
    adj = _build_adjacency(src, dst, edge_weight)
    A = adj.reshape(B, RP, RP)
    h1, h3 = _gcn_layers(A, xw1, b1, W3, b3)
    fc1w3 = fc1W.reshape(2 * ROIS, H1, D2)
    return _fc_head(h1, h3, fc1w3, fc1w3, fc1b, fc2W, fc2b, fc3W, fc3b,
                    bn1g, bn1b, bn2g, bn2b)
